# Initial kernel scaffold; baseline (speedup 1.0000x reference)
import jax
import jax.numpy as jnp
from jax.experimental import pallas as pl


def kernel(X, missing_mask, W_emb, b_emb, Wqk, Wv, Wo, ln1_g, ln1_b, W1, b1, W2, b2, ln2_g, ln2_b, W_out, b_out):
    raise NotImplementedError("write your pallas kernel here")


# speedup vs baseline: 2.2569x; 2.2569x over previous
import functools

import jax
import jax.numpy as jnp
import numpy as np
from jax.experimental import pallas as pl
from jax.experimental.pallas import tpu as pltpu

B, S, F, D, H, L = 2, 2048, 128, 1024, 16, 2
DH = D // H
DFF = 2048
BUCKET = 64
NHASH = 2
NB = S // BUCKET
NCH = S // BUCKET
RT = B * S
ROWT = 512
NRT = RT // ROWT
CH = 128
NCB = S // CH

_f32 = jnp.float32


def _fiota(shape, dim):
    return jax.lax.broadcasted_iota(jnp.int32, shape, dim).astype(_f32)


def _ln(x, g, b):
    m = jnp.mean(x, axis=1, keepdims=True)
    v = jnp.mean((x - m) ** 2, axis=1, keepdims=True)
    return (x - m) * jax.lax.rsqrt(v + 1e-5) * g + b


def _dot(a, b, dn=None):
    if dn is None:
        return jax.lax.dot(a, b, preferred_element_type=_f32)
    return jax.lax.dot_general(a, b, dimension_numbers=(dn, ((), ())),
                               preferred_element_type=_f32)


def _emb_body(x_ref, m_ref, wx_ref, wm_ref, b_ref, o_ref):
    o_ref[...] = (_dot(x_ref[...], wx_ref[...]) + _dot(m_ref[...], wm_ref[...])
                  + b_ref[...])


def _qkv_body(x_ref, wqk_ref, wv_ref, qk_ref, v_ref):
    x = x_ref[...]
    qk_ref[...] = _dot(x, wqk_ref[...])
    v_ref[...] = _dot(x, wv_ref[...])


def _block_body(a_ref, x_ref, wo_ref, g1_ref, b1g_ref, w1_ref, b1_ref,
                w2_ref, b2_ref, g2_ref, b2g_ref, o_ref):
    t = x_ref[...] + _dot(a_ref[...], wo_ref[...])
    t = _ln(t, g1_ref[...], b1g_ref[...])
    ff = jnp.maximum(_dot(t, w1_ref[...]) + b1_ref[...], 0.0)
    ff = _dot(ff, w2_ref[...]) + b2_ref[...]
    o_ref[...] = _ln(t + ff, g2_ref[...], b2g_ref[...])


def _final_body(x_ref, w_ref, b_ref, X_ref, m_ref, o_ref):
    rec = _dot(x_ref[...], w_ref[...]) + b_ref[...]
    m = m_ref[...]
    o_ref[...] = m * X_ref[...] + (1.0 - m) * rec


def _row_tiled(body, in_widths, out_widths):
    in_specs = []
    for w in in_widths:
        if isinstance(w, tuple) and w[0] == 't':
            in_specs.append(pl.BlockSpec((ROWT, w[1]), lambda i: (i, 0)))
        else:
            in_specs.append(pl.BlockSpec(w, lambda i: (0, 0)))
    out_specs = [pl.BlockSpec((ROWT, w), lambda i: (i, 0)) for w in out_widths]
    out_shape = [jax.ShapeDtypeStruct((RT, w), _f32) for w in out_widths]
    return pl.pallas_call(
        body,
        grid=(NRT,),
        in_specs=in_specs,
        out_specs=out_specs if len(out_widths) > 1 else out_specs[0],
        out_shape=out_shape if len(out_widths) > 1 else out_shape[0],
        compiler_params=pltpu.CompilerParams(
            dimension_semantics=("arbitrary",)),
    )


def _attn_body(qk_ref, v_ref, rot_ref, o_ref):
    qk = qk_ref[0]
    v = v_ref[0]

    lane128 = _fiota( (1, CH), 1)
    riota = _fiota( (S, 1), 0)
    r128 = _fiota( (CH, CH), 0)
    c128 = _fiota( (CH, CH), 1)
    Ls = jnp.where(c128 < r128, 1.0, 0.0)
    r16 = _fiota( (NCB, NCB), 0)
    c16 = _fiota( (NCB, NCB), 1)
    Ls16 = jnp.where(c16 < r16, 1.0, 0.0)
    r32 = _fiota( (NB, NB), 0)
    c32 = _fiota( (NB, NB), 1)
    Ts32 = jnp.where(r32 < c32, 1.0, 0.0)
    sub64 = _fiota( (BUCKET, 2 * BUCKET), 0)
    lan128 = _fiota( (BUCKET, 2 * BUCKET), 1)
    selfmask = lan128 == sub64 + float(BUCKET)

    vin = jnp.concatenate([qk, v], axis=1)

    per_round = []
    for r in range(NHASH):
        rot_r = rot_ref[r]
        rotated = _dot(qk, rot_r)
        full = jnp.concatenate([rotated, -rotated], axis=1)
        mx = jnp.max(full, axis=1, keepdims=True)
        iota_b = _fiota( (S, NB), 1)
        bucket = jnp.min(jnp.where(full >= mx, iota_b, 1e9), axis=1,
                         keepdims=True)

        onehot = jnp.where(iota_b == bucket, 1.0, 0.0)
        exs = []
        tots = []
        for c in range(NCB):
            blk = onehot[c * CH:(c + 1) * CH]
            exs.append(_dot(Ls, blk))
            tots.append(jnp.sum(blk, axis=0, keepdims=True))
        chunk_tot = jnp.concatenate(tots, axis=0)
        chunk_pre = _dot(Ls16, chunk_tot)
        wbp = jnp.concatenate(
            [exs[c] + chunk_pre[c:c + 1, :] for c in range(NCB)], axis=0)
        totals = jnp.sum(chunk_tot, axis=0, keepdims=True)
        offsets = _dot(totals, Ts32)
        rank = jnp.sum(onehot * (wbp + offsets), axis=1, keepdims=True)

        ends = offsets + totals
        son = jnp.where((riota >= offsets) & (riota < ends), 1.0, 0.0)

        schunks = []
        for c in range(NCB):
            pct = jnp.where(rank == float(c * CH) + lane128, 1.0, 0.0)
            schunks.append(_dot(pct, vin, dn=((0,), (0,))))
        svin = jnp.concatenate(schunks, axis=0)
        sqk = svin[:, 0:DH]
        sv = svin[:, DH:2 * DH]

        outs = []
        for a in range(NCH):
            p = (a - 1) % NCH
            q = sqk[a * BUCKET:(a + 1) * BUCKET]
            keys = jnp.concatenate([sqk[p * BUCKET:(p + 1) * BUCKET],
                                    q], axis=0)
            kn = keys * jax.lax.reciprocal(
                jnp.sqrt(jnp.sum(keys * keys, axis=1, keepdims=True)) + 1e-6)
            dots = _dot(q, kn, dn=((1,), (1,))) * (1.0 / np.sqrt(DH))
            son_q = son[a * BUCKET:(a + 1) * BUCKET]
            son_k = jnp.concatenate([son[p * BUCKET:(p + 1) * BUCKET],
                                     son_q], axis=0)
            eq = _dot(son_q, son_k, dn=((1,), (1,)))
            dots = jnp.where(eq > 0.5, dots, -1e9)
            dots = jnp.where(selfmask, -1e5, dots)
            m = jnp.max(dots, axis=1, keepdims=True)
            lse = m + jnp.log(jnp.sum(jnp.exp(dots - m), axis=1, keepdims=True))
            probs = jnp.exp(dots - lse)
            vals = jnp.concatenate([sv[p * BUCKET:(p + 1) * BUCKET],
                                    sv[a * BUCKET:(a + 1) * BUCKET]], axis=0)
            co = _dot(probs, vals)
            pad = jnp.zeros((BUCKET, CH - DH - 1), _f32)
            outs.append(jnp.concatenate([co, lse, pad], axis=1))
        so_ext = jnp.concatenate(outs, axis=0)

        o_uns = jnp.zeros((S, CH), _f32)
        for c in range(NCB):
            pct = jnp.where(rank == float(c * CH) + lane128, 1.0, 0.0)
            o_uns = o_uns + _dot(pct, so_ext[c * CH:(c + 1) * CH])
        per_round.append(o_uns)

    l0 = per_round[0][:, DH:DH + 1]
    l1 = per_round[1][:, DH:DH + 1]
    mm = jnp.maximum(l0, l1)
    e0 = jnp.exp(l0 - mm)
    e1 = jnp.exp(l1 - mm)
    o = (per_round[0][:, 0:DH] * e0 + per_round[1][:, 0:DH] * e1) \
        * jax.lax.reciprocal(e0 + e1)
    o_ref[0] = o


def _lsh_attention(qkh, vh, rot_i):
    return pl.pallas_call(
        _attn_body,
        grid=(B * H,),
        in_specs=[
            pl.BlockSpec((1, S, DH), lambda i: (i, 0, 0)),
            pl.BlockSpec((1, S, DH), lambda i: (i, 0, 0)),
            pl.BlockSpec((NHASH, DH, NB // 2), lambda i: (0, 0, 0)),
        ],
        out_specs=pl.BlockSpec((1, S, DH), lambda i: (i, 0, 0)),
        out_shape=jax.ShapeDtypeStruct((B * H, S, DH), _f32),
        compiler_params=pltpu.CompilerParams(
            dimension_semantics=("arbitrary",)),
    )(qkh, vh, rot_i)


def kernel(X, missing_mask, W_emb, b_emb, Wqk, Wv, Wo, ln1_g, ln1_b,
           W1, b1, W2, b2, ln2_g, ln2_b, W_out, b_out):
    X2 = X.reshape(RT, F)
    M2 = missing_mask.reshape(RT, F)

    emb = _row_tiled(_emb_body,
                     (('t', F), ('t', F), (F, D), (F, D), (1, D)), (D,))
    x = emb(X2, M2, W_emb[:F], W_emb[F:], b_emb.reshape(1, D))

    rot = jax.random.normal(jax.random.key(42), (L, DH, NHASH, NB // 2), _f32)

    qkv = _row_tiled(_qkv_body, (('t', D), (D, D), (D, D)), (D, D))
    blk = _row_tiled(_block_body,
                     (('t', D), ('t', D), (D, D), (1, D), (1, D),
                      (D, DFF), (1, DFF), (DFF, D), (1, D), (1, D), (1, D)),
                     (D,))

    for i in range(L):
        qk, v = qkv(x, Wqk[i], Wv[i])
        qkh = qk.reshape(B, S, H, DH).transpose(0, 2, 1, 3).reshape(B * H, S, DH)
        vh = v.reshape(B, S, H, DH).transpose(0, 2, 1, 3).reshape(B * H, S, DH)
        rot_i = rot[i].transpose(1, 0, 2)
        ah = _lsh_attention(qkh, vh, rot_i)
        a = ah.reshape(B, H, S, DH).transpose(0, 2, 1, 3).reshape(RT, D)
        x = blk(a, x, Wo[i], ln1_g[i].reshape(1, D), ln1_b[i].reshape(1, D),
                W1[i], b1[i].reshape(1, DFF), W2[i], b2[i].reshape(1, D),
                ln2_g[i].reshape(1, D), ln2_b[i].reshape(1, D))

    fin = pl.pallas_call(
        _final_body,
        grid=(NRT,),
        in_specs=[
            pl.BlockSpec((ROWT, D), lambda i: (i, 0)),
            pl.BlockSpec((D, F), lambda i: (0, 0)),
            pl.BlockSpec((1, F), lambda i: (0, 0)),
            pl.BlockSpec((ROWT, F), lambda i: (i, 0)),
            pl.BlockSpec((ROWT, F), lambda i: (i, 0)),
        ],
        out_specs=pl.BlockSpec((ROWT, F), lambda i: (i, 0)),
        out_shape=jax.ShapeDtypeStruct((RT, F), _f32),
        compiler_params=pltpu.CompilerParams(
            dimension_semantics=("arbitrary",)),
    )
    out = fin(x, W_out, b_out.reshape(1, F), X2, M2)
    return out.reshape(B, S, F)


# speedup vs baseline: 3.9252x; 1.7392x over previous
import functools

import jax
import jax.numpy as jnp
import numpy as np
from jax import lax
from jax.experimental import pallas as pl
from jax.experimental.pallas import tpu as pltpu
from jax.experimental.pallas import tpu_sc as plsc

B, S, F, D, H, L = 2, 2048, 128, 1024, 16, 2
DH = D // H
DFF = 2048
BUCKET = 64
NHASH = 2
NB = S // BUCKET
RT = B * S
ROWT = 512
NRT = RT // ROWT
CH = 128
NCB = S // CH
GC = 4
QR = GC * BUCKET
KR = QR + BUCKET
NG = S // QR
NTASK = B * H * NHASH
VROWS = B * H * S
GROWS = NTASK * S

_NC, _NS = 2, 16
_NW = _NC * _NS
_CHUNK = 128
_RPW = GROWS // _NW
_NSTEP = _RPW // _CHUNK

_f32 = jnp.float32


def _fiota(shape, dim):
    return lax.broadcasted_iota(jnp.int32, shape, dim).astype(_f32)


def _ln(x, g, b):
    m = jnp.mean(x, axis=1, keepdims=True)
    v = jnp.mean((x - m) ** 2, axis=1, keepdims=True)
    return (x - m) * lax.rsqrt(v + 1e-5) * g + b


def _dot(a, b, dn=None):
    if dn is None:
        return lax.dot(a, b, preferred_element_type=_f32)
    return lax.dot_general(a, b, dimension_numbers=(dn, ((), ())),
                           preferred_element_type=_f32)


def _emb_body(x_ref, m_ref, wx_ref, wm_ref, b_ref, o_ref):
    o_ref[...] = (_dot(x_ref[...], wx_ref[...]) + _dot(m_ref[...], wm_ref[...])
                  + b_ref[...])


def _qkv_body(x_ref, wqk_ref, wv_ref, qk_ref, v_ref):
    x = x_ref[...]
    qk_ref[...] = _dot(x, wqk_ref[...])
    v_ref[...] = _dot(x, wv_ref[...])


def _block_body(a_ref, x_ref, wo_ref, g1_ref, b1g_ref, w1_ref, b1_ref,
                w2_ref, b2_ref, g2_ref, b2g_ref, o_ref):
    t = x_ref[...] + _dot(a_ref[...], wo_ref[...])
    t = _ln(t, g1_ref[...], b1g_ref[...])
    ff = jnp.maximum(_dot(t, w1_ref[...]) + b1_ref[...], 0.0)
    ff = _dot(ff, w2_ref[...]) + b2_ref[...]
    o_ref[...] = _ln(t + ff, g2_ref[...], b2g_ref[...])


def _final_body(x_ref, w_ref, b_ref, X_ref, m_ref, o_ref):
    rec = _dot(x_ref[...], w_ref[...]) + b_ref[...]
    m = m_ref[...]
    o_ref[...] = m * X_ref[...] + (1.0 - m) * rec


def _row_tiled(body, in_widths, out_widths):
    in_specs = []
    for w in in_widths:
        if isinstance(w, tuple) and w[0] == 't':
            in_specs.append(pl.BlockSpec((ROWT, w[1]), lambda i: (i, 0)))
        else:
            in_specs.append(pl.BlockSpec(w, lambda i: (0, 0)))
    out_specs = [pl.BlockSpec((ROWT, w), lambda i: (i, 0)) for w in out_widths]
    out_shape = [jax.ShapeDtypeStruct((RT, w), _f32) for w in out_widths]
    return pl.pallas_call(
        body,
        grid=(NRT,),
        in_specs=in_specs,
        out_specs=out_specs if len(out_widths) > 1 else out_specs[0],
        out_shape=out_shape if len(out_widths) > 1 else out_shape[0],
        compiler_params=pltpu.CompilerParams(
            dimension_semantics=("arbitrary",)),
    )


def _prep_body(qk_ref, v_ref, rot_ref, vin_ref, guns_ref, oe_ref):
    pid = pl.program_id(0)
    qk = qk_ref[0]
    v = v_ref[0]
    vin_ref[0] = jnp.concatenate([qk, v], axis=1)

    riota = _fiota((S, 1), 0)
    r128 = _fiota((CH, CH), 0)
    c128 = _fiota((CH, CH), 1)
    Ls = jnp.where(c128 < r128, 1.0, 0.0)
    r16 = _fiota((NCB, NCB), 0)
    c16 = _fiota((NCB, NCB), 1)
    Ls16 = jnp.where(c16 < r16, 1.0, 0.0)
    r32 = _fiota((NB, NB), 0)
    c32 = _fiota((NB, NB), 1)
    Ts32 = jnp.where(r32 < c32, 1.0, 0.0)

    for r in range(NHASH):
        rotated = _dot(qk, rot_ref[r])
        full = jnp.concatenate([rotated, -rotated], axis=1)
        mx = jnp.max(full, axis=1, keepdims=True)
        iota_b = _fiota((S, NB), 1)
        bucket = jnp.min(jnp.where(full >= mx, iota_b, 1e9), axis=1,
                         keepdims=True)
        onehot = jnp.where(iota_b == bucket, 1.0, 0.0)
        exs = []
        tots = []
        for c in range(NCB):
            blk = onehot[c * CH:(c + 1) * CH]
            exs.append(_dot(Ls, blk))
            tots.append(jnp.sum(blk, axis=0, keepdims=True))
        chunk_tot = jnp.concatenate(tots, axis=0)
        chunk_pre = _dot(Ls16, chunk_tot)
        wbp = jnp.concatenate(
            [exs[c] + chunk_pre[c:c + 1, :] for c in range(NCB)], axis=0)
        totals = jnp.sum(chunk_tot, axis=0, keepdims=True)
        offsets = _dot(totals, Ts32)
        rank = jnp.sum(onehot * (wbp + offsets), axis=1, keepdims=True)

        base = (pid * NHASH + r) * S
        guns_ref[0, r] = (rank + base.astype(_f32)).astype(jnp.int32)
        oe_ref[0, r] = jnp.concatenate([offsets, offsets + totals], axis=1)


def _prep(qkh, vh, rot_i):
    return pl.pallas_call(
        _prep_body,
        grid=(B * H,),
        in_specs=[
            pl.BlockSpec((1, S, DH), lambda i: (i, 0, 0)),
            pl.BlockSpec((1, S, DH), lambda i: (i, 0, 0)),
            pl.BlockSpec((NHASH, DH, NB // 2), lambda i: (0, 0, 0)),
        ],
        out_specs=[
            pl.BlockSpec((1, S, 2 * DH), lambda i: (i, 0, 0)),
            pl.BlockSpec((1, NHASH, S, 1), lambda i: (i, 0, 0, 0)),
            pl.BlockSpec((1, NHASH, 1, 2 * NB), lambda i: (i, 0, 0, 0)),
        ],
        out_shape=[
            jax.ShapeDtypeStruct((B * H, S, 2 * DH), _f32),
            jax.ShapeDtypeStruct((B * H, NHASH, S, 1), jnp.int32),
            jax.ShapeDtypeStruct((B * H, NHASH, 1, 2 * NB), _f32),
        ],
        compiler_params=pltpu.CompilerParams(
            dimension_semantics=("arbitrary",)),
    )(qkh, vh, rot_i)


def _sc_sort(vin_flat, gidx):
    @functools.partial(
        pl.kernel,
        mesh=plsc.VectorSubcoreMesh(core_axis_name="c", subcore_axis_name="s"),
        out_type=jax.ShapeDtypeStruct((GROWS, 2 * DH), _f32),
        scratch_types=[
            pltpu.VMEM((_CHUNK,), jnp.int32),
            pltpu.VMEM((_CHUNK, 2 * DH), _f32),
            pltpu.SemaphoreType.DMA,
        ],
    )
    def k(vin_hbm, idx_hbm, out_hbm, idx_v, rows_v, sem):
        wid = lax.axis_index("s") * _NC + lax.axis_index("c")

        def step(j, carry):
            g0 = wid * _RPW + j * _CHUNK
            t = g0 // S
            src0 = (t // NHASH) * S + (g0 - t * S)
            pltpu.sync_copy(vin_hbm.at[pl.ds(src0, _CHUNK)], rows_v)
            pltpu.sync_copy(idx_hbm.at[pl.ds(g0, _CHUNK)], idx_v)
            pltpu.async_copy(rows_v, out_hbm.at[idx_v], sem).wait()
            return carry

        lax.fori_loop(0, _NSTEP, step, 0)

    return k(vin_flat, gidx)


def _sc_unsort(so_flat, gidx):
    @functools.partial(
        pl.kernel,
        mesh=plsc.VectorSubcoreMesh(core_axis_name="c", subcore_axis_name="s"),
        out_type=jax.ShapeDtypeStruct((GROWS, CH), _f32),
        scratch_types=[
            pltpu.VMEM((_CHUNK,), jnp.int32),
            pltpu.VMEM((_CHUNK, CH), _f32),
            pltpu.SemaphoreType.DMA,
        ],
    )
    def k(so_hbm, idx_hbm, out_hbm, idx_v, rows_v, sem):
        wid = lax.axis_index("s") * _NC + lax.axis_index("c")

        def step(j, carry):
            g0 = wid * _RPW + j * _CHUNK
            pltpu.sync_copy(idx_hbm.at[pl.ds(g0, _CHUNK)], idx_v)
            pltpu.async_copy(so_hbm.at[idx_v], rows_v, sem).wait()
            pltpu.sync_copy(rows_v, out_hbm.at[pl.ds(g0, _CHUNK)])
            return carry

        lax.fori_loop(0, _NSTEP, step, 0)

    return k(so_flat, gidx)


def _attn2_body(svin_ref, oe_ref, so_ref):
    sqk = svin_ref[0, :, 0:DH]
    sv = svin_ref[0, :, DH:2 * DH]
    offsets = oe_ref[0, :, 0:NB]
    ends = oe_ref[0, :, NB:2 * NB]

    riota = _fiota((S, 1), 0)
    son = jnp.where((riota >= offsets) & (riota < ends), 1.0, 0.0)

    kn = sqk * lax.reciprocal(
        jnp.sqrt(jnp.sum(sqk * sqk, axis=1, keepdims=True)) + 1e-6)

    si = lax.broadcasted_iota(jnp.int32, (QR, KR), 0)
    li = lax.broadcasted_iota(jnp.int32, (QR, KR), 1)
    cbase = (si // BUCKET) * BUCKET
    band = (li >= cbase) & (li < cbase + 2 * BUCKET)
    selfm = li == si + BUCKET
    scale = 1.0 / np.sqrt(DH)
    pad = jnp.zeros((QR, CH - DH - 1), _f32)

    def kseg(arr, g):
        if g == 0:
            return jnp.concatenate([arr[S - BUCKET:S], arr[0:QR]], axis=0)
        return arr[g * QR - BUCKET:g * QR + QR]

    outs = []
    for g in range(NG):
        q = sqk[g * QR:(g + 1) * QR]
        dots = _dot(q, kseg(kn, g), dn=((1,), (1,))) * scale
        eq = _dot(son[g * QR:(g + 1) * QR], kseg(son, g),
                  dn=((1,), (1,)))
        dots = jnp.where((eq > 0.5) & band, dots, -1e9)
        dots = jnp.where(selfm, -1e5, dots)
        m = jnp.max(dots, axis=1, keepdims=True)
        lse = m + jnp.log(jnp.sum(jnp.exp(dots - m), axis=1, keepdims=True))
        probs = jnp.exp(dots - lse)
        co = _dot(probs, kseg(sv, g))
        outs.append(jnp.concatenate([co, lse, pad], axis=1))
    so_ref[0] = jnp.concatenate(outs, axis=0)


def _attn2(svin, oe2):
    return pl.pallas_call(
        _attn2_body,
        grid=(NTASK,),
        in_specs=[
            pl.BlockSpec((1, S, 2 * DH), lambda i: (i, 0, 0)),
            pl.BlockSpec((1, 1, 2 * NB), lambda i: (i, 0, 0)),
        ],
        out_specs=pl.BlockSpec((1, S, CH), lambda i: (i, 0, 0)),
        out_shape=jax.ShapeDtypeStruct((NTASK, S, CH), _f32),
        compiler_params=pltpu.CompilerParams(
            dimension_semantics=("arbitrary",)),
    )(svin, oe2)


def _comb_body(ouns_ref, ah_ref):
    o0 = ouns_ref[0, 0:S, 0:DH]
    l0 = ouns_ref[0, 0:S, DH:DH + 1]
    o1 = ouns_ref[0, S:2 * S, 0:DH]
    l1 = ouns_ref[0, S:2 * S, DH:DH + 1]
    mm = jnp.maximum(l0, l1)
    e0 = jnp.exp(l0 - mm)
    e1 = jnp.exp(l1 - mm)
    ah_ref[0] = (o0 * e0 + o1 * e1) * lax.reciprocal(e0 + e1)


def _comb(ouns2):
    return pl.pallas_call(
        _comb_body,
        grid=(B * H,),
        in_specs=[pl.BlockSpec((1, NHASH * S, CH), lambda i: (i, 0, 0))],
        out_specs=pl.BlockSpec((1, S, DH), lambda i: (i, 0, 0)),
        out_shape=jax.ShapeDtypeStruct((B * H, S, DH), _f32),
        compiler_params=pltpu.CompilerParams(
            dimension_semantics=("arbitrary",)),
    )(ouns2)


def kernel(X, missing_mask, W_emb, b_emb, Wqk, Wv, Wo, ln1_g, ln1_b,
           W1, b1, W2, b2, ln2_g, ln2_b, W_out, b_out):
    X2 = X.reshape(RT, F)
    M2 = missing_mask.reshape(RT, F)

    emb = _row_tiled(_emb_body,
                     (('t', F), ('t', F), (F, D), (F, D), (1, D)), (D,))
    x = emb(X2, M2, W_emb[:F], W_emb[F:], b_emb.reshape(1, D))

    rot = jax.random.normal(jax.random.key(42), (L, DH, NHASH, NB // 2), _f32)

    qkv = _row_tiled(_qkv_body, (('t', D), (D, D), (D, D)), (D, D))
    blk = _row_tiled(_block_body,
                     (('t', D), ('t', D), (D, D), (1, D), (1, D),
                      (D, DFF), (1, DFF), (DFF, D), (1, D), (1, D), (1, D)),
                     (D,))

    for i in range(L):
        qk, v = qkv(x, Wqk[i], Wv[i])
        qkh = qk.reshape(B, S, H, DH).transpose(0, 2, 1, 3).reshape(B * H, S, DH)
        vh = v.reshape(B, S, H, DH).transpose(0, 2, 1, 3).reshape(B * H, S, DH)
        rot_i = rot[i].transpose(1, 0, 2)

        vin, guns4, oe4 = _prep(qkh, vh, rot_i)
        guns = guns4.reshape(GROWS)
        svin = _sc_sort(vin.reshape(VROWS, 2 * DH), guns)
        so = _attn2(svin.reshape(NTASK, S, 2 * DH),
                    oe4.reshape(NTASK, 1, 2 * NB))
        ouns = _sc_unsort(so.reshape(GROWS, CH), guns)
        ah = _comb(ouns.reshape(B * H, NHASH * S, CH))

        a = ah.reshape(B, H, S, DH).transpose(0, 2, 1, 3).reshape(RT, D)
        x = blk(a, x, Wo[i], ln1_g[i].reshape(1, D), ln1_b[i].reshape(1, D),
                W1[i], b1[i].reshape(1, DFF), W2[i], b2[i].reshape(1, D),
                ln2_g[i].reshape(1, D), ln2_b[i].reshape(1, D))

    fin = pl.pallas_call(
        _final_body,
        grid=(NRT,),
        in_specs=[
            pl.BlockSpec((ROWT, D), lambda i: (i, 0)),
            pl.BlockSpec((D, F), lambda i: (0, 0)),
            pl.BlockSpec((1, F), lambda i: (0, 0)),
            pl.BlockSpec((ROWT, F), lambda i: (i, 0)),
            pl.BlockSpec((ROWT, F), lambda i: (i, 0)),
        ],
        out_specs=pl.BlockSpec((ROWT, F), lambda i: (i, 0)),
        out_shape=jax.ShapeDtypeStruct((RT, F), _f32),
        compiler_params=pltpu.CompilerParams(
            dimension_semantics=("arbitrary",)),
    )
    out = fin(x, W_out, b_out.reshape(1, F), X2, M2)
    return out.reshape(B, S, F)


# speedup vs baseline: 3.9398x; 1.0037x over previous
import functools

import jax
import jax.numpy as jnp
import numpy as np
from jax import lax
from jax.experimental import pallas as pl
from jax.experimental.pallas import tpu as pltpu
from jax.experimental.pallas import tpu_sc as plsc

B, S, F, D, H, L = 2, 2048, 128, 1024, 16, 2
DH = D // H
DFF = 2048
BUCKET = 64
NHASH = 2
NB = S // BUCKET
RT = B * S
ROWT = 512
NRT = RT // ROWT
NST = S // ROWT
CH = 128
NCB = S // CH
GC = 2
QR = GC * BUCKET
KR = QR + BUCKET
NG = S // QR
NTASK = B * H * NHASH
VROWS = B * H * S
GROWS = NTASK * S

_NC, _NS = 2, 16
_NW = _NC * _NS
_CHUNK = 128
_RPW = GROWS // _NW
_NSTEP = _RPW // _CHUNK

_f32 = jnp.float32


def _fiota(shape, dim):
    return lax.broadcasted_iota(jnp.int32, shape, dim).astype(_f32)


def _ln(x, g, b):
    m = jnp.mean(x, axis=1, keepdims=True)
    v = jnp.mean((x - m) ** 2, axis=1, keepdims=True)
    return (x - m) * lax.rsqrt(v + 1e-5) * g + b


def _dot(a, b, dn=None):
    if dn is None:
        return lax.dot(a, b, preferred_element_type=_f32)
    return lax.dot_general(a, b, dimension_numbers=(dn, ((), ())),
                           preferred_element_type=_f32)


def _emb_body(x_ref, m_ref, wx_ref, wm_ref, b_ref, o_ref):
    o_ref[...] = (_dot(x_ref[...], wx_ref[...]) + _dot(m_ref[...], wm_ref[...])
                  + b_ref[...])


def _qkv_body(x_ref, wqk_ref, wv_ref, qk_ref, v_ref):
    x = x_ref[...]
    qk = _dot(x, wqk_ref[...])
    v = _dot(x, wv_ref[...])
    for h in range(H):
        qk_ref[0, h] = qk[:, h * DH:(h + 1) * DH]
        v_ref[0, h] = v[:, h * DH:(h + 1) * DH]


def _block_body(a_ref, x_ref, wo_ref, g1_ref, b1g_ref, w1_ref, b1_ref,
                w2_ref, b2_ref, g2_ref, b2g_ref, o_ref):
    a = jnp.concatenate([a_ref[0, h] for h in range(H)], axis=1)
    t = x_ref[...] + _dot(a, wo_ref[...])
    t = _ln(t, g1_ref[...], b1g_ref[...])
    ff = jnp.maximum(_dot(t, w1_ref[...]) + b1_ref[...], 0.0)
    ff = _dot(ff, w2_ref[...]) + b2_ref[...]
    o_ref[...] = _ln(t + ff, g2_ref[...], b2g_ref[...])


def _final_body(x_ref, w_ref, b_ref, X_ref, m_ref, o_ref):
    rec = _dot(x_ref[...], w_ref[...]) + b_ref[...]
    m = m_ref[...]
    o_ref[...] = m * X_ref[...] + (1.0 - m) * rec


def _row_tiled(body, in_widths, out_widths):
    in_specs = []
    for w in in_widths:
        if isinstance(w, tuple) and w[0] == 't':
            in_specs.append(pl.BlockSpec((ROWT, w[1]), lambda i: (i, 0)))
        else:
            in_specs.append(pl.BlockSpec(w, lambda i: (0, 0)))
    out_specs = [pl.BlockSpec((ROWT, w), lambda i: (i, 0)) for w in out_widths]
    out_shape = [jax.ShapeDtypeStruct((RT, w), _f32) for w in out_widths]
    return pl.pallas_call(
        body,
        grid=(NRT,),
        in_specs=in_specs,
        out_specs=out_specs if len(out_widths) > 1 else out_specs[0],
        out_shape=out_shape if len(out_widths) > 1 else out_shape[0],
        compiler_params=pltpu.CompilerParams(
            dimension_semantics=("arbitrary",)),
    )


def _prep_body(qk_ref, v_ref, rot_ref, vin_ref, guns_ref, oe_ref):
    pid = pl.program_id(0)
    qk = qk_ref[0, 0]
    v = v_ref[0, 0]
    vin_ref[0] = jnp.concatenate([qk, v], axis=1)

    riota = _fiota((S, 1), 0)
    r128 = _fiota((CH, CH), 0)
    c128 = _fiota((CH, CH), 1)
    Ls = jnp.where(c128 < r128, 1.0, 0.0)
    r32 = _fiota((NB, NB), 0)
    c32 = _fiota((NB, NB), 1)
    Ts32 = jnp.where(r32 < c32, 1.0, 0.0)

    for r in range(NHASH):
        rotated = _dot(qk, rot_ref[r])
        full = jnp.concatenate([rotated, -rotated], axis=1)
        mx = jnp.max(full, axis=1, keepdims=True)
        iota_b = _fiota((S, NB), 1)
        bucket = jnp.min(jnp.where(full >= mx, iota_b, 1e9), axis=1,
                         keepdims=True)
        onehot = jnp.where(iota_b == bucket, 1.0, 0.0)
        blkall = jnp.concatenate(
            [onehot[c * CH:(c + 1) * CH] for c in range(NCB)], axis=1)
        exall = _dot(Ls, blkall)
        lastrow = exall[CH - 1:CH, :] + blkall[CH - 1:CH, :]
        wbps = []
        run = jnp.zeros((1, NB), _f32)
        for c in range(NCB):
            sl = slice(c * NB, (c + 1) * NB)
            wbps.append(exall[:, sl] + run)
            run = run + lastrow[:, sl]
        wbp = jnp.concatenate(wbps, axis=0)
        totals = run
        offsets = _dot(totals, Ts32)
        rank = jnp.sum(onehot * (wbp + offsets), axis=1, keepdims=True)

        base = (pid * NHASH + r) * S
        guns_ref[0, r] = (rank + base.astype(_f32)).astype(jnp.int32)
        oe_ref[0, r] = jnp.concatenate([offsets, offsets + totals], axis=1)


def _prep(qkh, vh, rot_i):
    return pl.pallas_call(
        _prep_body,
        grid=(B * H,),
        in_specs=[
            pl.BlockSpec((1, 1, S, DH), lambda i: (i // H, i % H, 0, 0)),
            pl.BlockSpec((1, 1, S, DH), lambda i: (i // H, i % H, 0, 0)),
            pl.BlockSpec((NHASH, DH, NB // 2), lambda i: (0, 0, 0)),
        ],
        out_specs=[
            pl.BlockSpec((1, S, 2 * DH), lambda i: (i, 0, 0)),
            pl.BlockSpec((1, NHASH, S, 1), lambda i: (i, 0, 0, 0)),
            pl.BlockSpec((1, NHASH, 1, 2 * NB), lambda i: (i, 0, 0, 0)),
        ],
        out_shape=[
            jax.ShapeDtypeStruct((B * H, S, 2 * DH), _f32),
            jax.ShapeDtypeStruct((B * H, NHASH, S, 1), jnp.int32),
            jax.ShapeDtypeStruct((B * H, NHASH, 1, 2 * NB), _f32),
        ],
        compiler_params=pltpu.CompilerParams(
            dimension_semantics=("arbitrary",)),
    )(qkh, vh, rot_i)


def _sc_sort(vin_flat, gidx):
    @functools.partial(
        pl.kernel,
        mesh=plsc.VectorSubcoreMesh(core_axis_name="c", subcore_axis_name="s"),
        out_type=jax.ShapeDtypeStruct((GROWS, 2 * DH), _f32),
        scratch_types=[
            pltpu.VMEM((_CHUNK,), jnp.int32),
            pltpu.VMEM((_CHUNK, 2 * DH), _f32),
            pltpu.SemaphoreType.DMA,
        ],
    )
    def k(vin_hbm, idx_hbm, out_hbm, idx_v, rows_v, sem):
        wid = lax.axis_index("s") * _NC + lax.axis_index("c")

        def step(j, carry):
            g0 = wid * _RPW + j * _CHUNK
            t = g0 // S
            src0 = (t // NHASH) * S + (g0 - t * S)
            pltpu.sync_copy(vin_hbm.at[pl.ds(src0, _CHUNK)], rows_v)
            pltpu.sync_copy(idx_hbm.at[pl.ds(g0, _CHUNK)], idx_v)
            pltpu.async_copy(rows_v, out_hbm.at[idx_v], sem).wait()
            return carry

        lax.fori_loop(0, _NSTEP, step, 0)

    return k(vin_flat, gidx)


def _sc_unsort(so_flat, gidx):
    @functools.partial(
        pl.kernel,
        mesh=plsc.VectorSubcoreMesh(core_axis_name="c", subcore_axis_name="s"),
        out_type=jax.ShapeDtypeStruct((GROWS, CH), _f32),
        scratch_types=[
            pltpu.VMEM((_CHUNK,), jnp.int32),
            pltpu.VMEM((_CHUNK, CH), _f32),
            pltpu.SemaphoreType.DMA,
        ],
    )
    def k(so_hbm, idx_hbm, out_hbm, idx_v, rows_v, sem):
        wid = lax.axis_index("s") * _NC + lax.axis_index("c")

        def step(j, carry):
            g0 = wid * _RPW + j * _CHUNK
            pltpu.sync_copy(idx_hbm.at[pl.ds(g0, _CHUNK)], idx_v)
            pltpu.async_copy(so_hbm.at[idx_v], rows_v, sem).wait()
            pltpu.sync_copy(rows_v, out_hbm.at[pl.ds(g0, _CHUNK)])
            return carry

        lax.fori_loop(0, _NSTEP, step, 0)

    return k(so_flat, gidx)


def _attn2_body(svin_ref, oe_ref, so_ref):
    sqk = svin_ref[0, :, 0:DH]
    sv = svin_ref[0, :, DH:2 * DH]
    offsets = oe_ref[0, :, 0:NB]
    ends = oe_ref[0, :, NB:2 * NB]

    riota = _fiota((S, 1), 0)
    son = jnp.where((riota >= offsets) & (riota < ends), 1.0, 0.0)

    kn = sqk * lax.reciprocal(
        jnp.sqrt(jnp.sum(sqk * sqk, axis=1, keepdims=True)) + 1e-6)

    si = lax.broadcasted_iota(jnp.int32, (QR, KR), 0)
    li = lax.broadcasted_iota(jnp.int32, (QR, KR), 1)
    cbase = (si // BUCKET) * BUCKET
    band = (li >= cbase) & (li < cbase + 2 * BUCKET)
    selfm = li == si + BUCKET
    scale = 1.0 / np.sqrt(DH)
    pad = jnp.zeros((QR, CH - DH - 1), _f32)

    def kseg(arr, g):
        if g == 0:
            return jnp.concatenate([arr[S - BUCKET:S], arr[0:QR]], axis=0)
        return arr[g * QR - BUCKET:g * QR + QR]

    outs = []
    for g in range(NG):
        q = sqk[g * QR:(g + 1) * QR]
        dots = _dot(q, kseg(kn, g), dn=((1,), (1,))) * scale
        eq = _dot(son[g * QR:(g + 1) * QR], kseg(son, g),
                  dn=((1,), (1,)))
        dots = jnp.where((eq > 0.5) & band, dots, -1e9)
        dots = jnp.where(selfm, -1e5, dots)
        m = jnp.max(dots, axis=1, keepdims=True)
        lse = m + jnp.log(jnp.sum(jnp.exp(dots - m), axis=1, keepdims=True))
        probs = jnp.exp(dots - lse)
        co = _dot(probs, kseg(sv, g))
        outs.append(jnp.concatenate([co, lse, pad], axis=1))
    so_ref[0] = jnp.concatenate(outs, axis=0)


def _attn2(svin, oe2):
    return pl.pallas_call(
        _attn2_body,
        grid=(NTASK,),
        in_specs=[
            pl.BlockSpec((1, S, 2 * DH), lambda i: (i, 0, 0)),
            pl.BlockSpec((1, 1, 2 * NB), lambda i: (i, 0, 0)),
        ],
        out_specs=pl.BlockSpec((1, S, CH), lambda i: (i, 0, 0)),
        out_shape=jax.ShapeDtypeStruct((NTASK, S, CH), _f32),
        compiler_params=pltpu.CompilerParams(
            dimension_semantics=("arbitrary",)),
    )(svin, oe2)


def _comb_body(ouns_ref, ah_ref):
    o0 = ouns_ref[0, 0:S, 0:DH]
    l0 = ouns_ref[0, 0:S, DH:DH + 1]
    o1 = ouns_ref[0, S:2 * S, 0:DH]
    l1 = ouns_ref[0, S:2 * S, DH:DH + 1]
    mm = jnp.maximum(l0, l1)
    e0 = jnp.exp(l0 - mm)
    e1 = jnp.exp(l1 - mm)
    ah_ref[0, 0] = (o0 * e0 + o1 * e1) * lax.reciprocal(e0 + e1)


def _comb(ouns2):
    return pl.pallas_call(
        _comb_body,
        grid=(B * H,),
        in_specs=[pl.BlockSpec((1, NHASH * S, CH), lambda i: (i, 0, 0))],
        out_specs=pl.BlockSpec((1, 1, S, DH),
                               lambda i: (i // H, i % H, 0, 0)),
        out_shape=jax.ShapeDtypeStruct((B, H, S, DH), _f32),
        compiler_params=pltpu.CompilerParams(
            dimension_semantics=("arbitrary",)),
    )(ouns2)


def kernel(X, missing_mask, W_emb, b_emb, Wqk, Wv, Wo, ln1_g, ln1_b,
           W1, b1, W2, b2, ln2_g, ln2_b, W_out, b_out):
    X2 = X.reshape(RT, F)
    M2 = missing_mask.reshape(RT, F)

    emb = _row_tiled(_emb_body,
                     (('t', F), ('t', F), (F, D), (F, D), (1, D)), (D,))
    x = emb(X2, M2, W_emb[:F], W_emb[F:], b_emb.reshape(1, D))

    rot = jax.random.normal(jax.random.key(42), (L, DH, NHASH, NB // 2), _f32)

    hspec = pl.BlockSpec((1, H, ROWT, DH), lambda i: (i // NST, 0, i % NST, 0))
    qkv = pl.pallas_call(
        _qkv_body,
        grid=(NRT,),
        in_specs=[
            pl.BlockSpec((ROWT, D), lambda i: (i, 0)),
            pl.BlockSpec((D, D), lambda i: (0, 0)),
            pl.BlockSpec((D, D), lambda i: (0, 0)),
        ],
        out_specs=[hspec, hspec],
        out_shape=[jax.ShapeDtypeStruct((B, H, S, DH), _f32)] * 2,
        compiler_params=pltpu.CompilerParams(
            dimension_semantics=("arbitrary",)),
    )
    blk = pl.pallas_call(
        _block_body,
        grid=(NRT,),
        in_specs=[hspec,
                  pl.BlockSpec((ROWT, D), lambda i: (i, 0)),
                  pl.BlockSpec((D, D), lambda i: (0, 0)),
                  pl.BlockSpec((1, D), lambda i: (0, 0)),
                  pl.BlockSpec((1, D), lambda i: (0, 0)),
                  pl.BlockSpec((D, DFF), lambda i: (0, 0)),
                  pl.BlockSpec((1, DFF), lambda i: (0, 0)),
                  pl.BlockSpec((DFF, D), lambda i: (0, 0)),
                  pl.BlockSpec((1, D), lambda i: (0, 0)),
                  pl.BlockSpec((1, D), lambda i: (0, 0)),
                  pl.BlockSpec((1, D), lambda i: (0, 0))],
        out_specs=pl.BlockSpec((ROWT, D), lambda i: (i, 0)),
        out_shape=jax.ShapeDtypeStruct((RT, D), _f32),
        compiler_params=pltpu.CompilerParams(
            dimension_semantics=("arbitrary",)),
    )

    for i in range(L):
        qkh4, vh4 = qkv(x, Wqk[i], Wv[i])
        rot_i = rot[i].transpose(1, 0, 2)

        vin, guns4, oe4 = _prep(qkh4, vh4, rot_i)
        guns = guns4.reshape(GROWS)
        svin = _sc_sort(vin.reshape(VROWS, 2 * DH), guns)
        so = _attn2(svin.reshape(NTASK, S, 2 * DH),
                    oe4.reshape(NTASK, 1, 2 * NB))
        ouns = _sc_unsort(so.reshape(GROWS, CH), guns)
        ah4 = _comb(ouns.reshape(B * H, NHASH * S, CH))

        x = blk(ah4, x, Wo[i], ln1_g[i].reshape(1, D), ln1_b[i].reshape(1, D),
                W1[i], b1[i].reshape(1, DFF), W2[i], b2[i].reshape(1, D),
                ln2_g[i].reshape(1, D), ln2_b[i].reshape(1, D))

    fin = pl.pallas_call(
        _final_body,
        grid=(NRT,),
        in_specs=[
            pl.BlockSpec((ROWT, D), lambda i: (i, 0)),
            pl.BlockSpec((D, F), lambda i: (0, 0)),
            pl.BlockSpec((1, F), lambda i: (0, 0)),
            pl.BlockSpec((ROWT, F), lambda i: (i, 0)),
            pl.BlockSpec((ROWT, F), lambda i: (i, 0)),
        ],
        out_specs=pl.BlockSpec((ROWT, F), lambda i: (i, 0)),
        out_shape=jax.ShapeDtypeStruct((RT, F), _f32),
        compiler_params=pltpu.CompilerParams(
            dimension_semantics=("arbitrary",)),
    )
    out = fin(x, W_out, b_out.reshape(1, F), X2, M2)
    return out.reshape(B, S, F)


# speedup vs baseline: 4.7390x; 1.2029x over previous
import functools

import jax
import jax.numpy as jnp
import numpy as np
from jax import lax
from jax.experimental import pallas as pl
from jax.experimental.pallas import tpu as pltpu
from jax.experimental.pallas import tpu_sc as plsc

B, S, F, D, H, L = 2, 2048, 128, 1024, 16, 2
DH = D // H
DFF = 2048
BUCKET = 64
NHASH = 2
NB = S // BUCKET
RT = B * S
ROWT = 512
NRT = RT // ROWT
NST = S // ROWT
CH = 128
NCB = S // CH
GC = 4
QR = GC * BUCKET
KR = QR + BUCKET
NG = S // QR
NTASK = B * H * NHASH
VROWS = B * H * S
GROWS = NTASK * S

_NC, _NS = 2, 16
_NW = _NC * _NS
_CHUNK = 128
_RPW = GROWS // _NW
_NSTEP = _RPW // _CHUNK

_f32 = jnp.float32


def _fiota(shape, dim):
    return lax.broadcasted_iota(jnp.int32, shape, dim).astype(_f32)


def _ln(x, g, b):
    m = jnp.mean(x, axis=1, keepdims=True)
    v = jnp.mean((x - m) ** 2, axis=1, keepdims=True)
    return (x - m) * lax.rsqrt(v + 1e-5) * g + b


def _dot(a, b, dn=None):
    if dn is None:
        return lax.dot(a, b, preferred_element_type=_f32)
    return lax.dot_general(a, b, dimension_numbers=(dn, ((), ())),
                           preferred_element_type=_f32)


def _emb_body(x_ref, m_ref, wx_ref, wm_ref, b_ref, o_ref):
    o_ref[...] = (_dot(x_ref[...], wx_ref[...]) + _dot(m_ref[...], wm_ref[...])
                  + b_ref[...])


def _qkv_body(x_ref, wqk_ref, wv_ref, qk_ref, v_ref):
    x = x_ref[...]
    qk = _dot(x, wqk_ref[...])
    v = _dot(x, wv_ref[...])
    for h in range(H):
        qk_ref[0, h] = qk[:, h * DH:(h + 1) * DH]
        v_ref[0, h] = v[:, h * DH:(h + 1) * DH]


def _block_body(a_ref, x_ref, wo_ref, g1_ref, b1g_ref, w1_ref, b1_ref,
                w2_ref, b2_ref, g2_ref, b2g_ref, o_ref):
    a = jnp.concatenate([a_ref[0, h] for h in range(H)], axis=1)
    t = x_ref[...] + _dot(a, wo_ref[...])
    t = _ln(t, g1_ref[...], b1g_ref[...])
    ff = jnp.maximum(_dot(t, w1_ref[...]) + b1_ref[...], 0.0)
    ff = _dot(ff, w2_ref[...]) + b2_ref[...]
    o_ref[...] = _ln(t + ff, g2_ref[...], b2g_ref[...])


def _final_body(x_ref, w_ref, b_ref, X_ref, m_ref, o_ref):
    rec = _dot(x_ref[...], w_ref[...]) + b_ref[...]
    m = m_ref[...]
    o_ref[...] = m * X_ref[...] + (1.0 - m) * rec


def _row_tiled(body, in_widths, out_widths):
    in_specs = []
    for w in in_widths:
        if isinstance(w, tuple) and w[0] == 't':
            in_specs.append(pl.BlockSpec((ROWT, w[1]), lambda i: (i, 0)))
        else:
            in_specs.append(pl.BlockSpec(w, lambda i: (0, 0)))
    out_specs = [pl.BlockSpec((ROWT, w), lambda i: (i, 0)) for w in out_widths]
    out_shape = [jax.ShapeDtypeStruct((RT, w), _f32) for w in out_widths]
    return pl.pallas_call(
        body,
        grid=(NRT,),
        in_specs=in_specs,
        out_specs=out_specs if len(out_widths) > 1 else out_specs[0],
        out_shape=out_shape if len(out_widths) > 1 else out_shape[0],
        compiler_params=pltpu.CompilerParams(
            dimension_semantics=("arbitrary",)),
    )


def _prep_body(qk_ref, v_ref, rot_ref, vin_ref, guns_ref, oe_ref):
    pid = pl.program_id(0)
    qk = qk_ref[0, 0]
    v = v_ref[0, 0]
    vin_ref[0] = jnp.concatenate([qk, v], axis=1)

    riota = _fiota((S, 1), 0)
    r128 = _fiota((CH, CH), 0)
    c128 = _fiota((CH, CH), 1)
    Ls = jnp.where(c128 < r128, 1.0, 0.0)
    r32 = _fiota((NB, NB), 0)
    c32 = _fiota((NB, NB), 1)
    Ts32 = jnp.where(r32 < c32, 1.0, 0.0)

    for r in range(NHASH):
        rotated = _dot(qk, rot_ref[r])
        full = jnp.concatenate([rotated, -rotated], axis=1)
        mx = jnp.max(full, axis=1, keepdims=True)
        iota_b = _fiota((S, NB), 1)
        bucket = jnp.min(jnp.where(full >= mx, iota_b, 1e9), axis=1,
                         keepdims=True)
        onehot = jnp.where(iota_b == bucket, 1.0, 0.0)
        blkall = jnp.concatenate(
            [onehot[c * CH:(c + 1) * CH] for c in range(NCB)], axis=1)
        exall = _dot(Ls, blkall)
        lastrow = exall[CH - 1:CH, :] + blkall[CH - 1:CH, :]
        wbps = []
        run = jnp.zeros((1, NB), _f32)
        for c in range(NCB):
            sl = slice(c * NB, (c + 1) * NB)
            wbps.append(exall[:, sl] + run)
            run = run + lastrow[:, sl]
        wbp = jnp.concatenate(wbps, axis=0)
        totals = run
        offsets = _dot(totals, Ts32)
        rank = jnp.sum(onehot * (wbp + offsets), axis=1, keepdims=True)

        base = (pid * NHASH + r) * S
        guns_ref[0, r] = (rank + base.astype(_f32)).astype(jnp.int32)
        oe_ref[0, r] = jnp.concatenate([offsets, offsets + totals], axis=1)


def _prep(qkh, vh, rot_i):
    return pl.pallas_call(
        _prep_body,
        grid=(B * H,),
        in_specs=[
            pl.BlockSpec((1, 1, S, DH), lambda i: (i // H, i % H, 0, 0)),
            pl.BlockSpec((1, 1, S, DH), lambda i: (i // H, i % H, 0, 0)),
            pl.BlockSpec((NHASH, DH, NB // 2), lambda i: (0, 0, 0)),
        ],
        out_specs=[
            pl.BlockSpec((1, S, 2 * DH), lambda i: (i, 0, 0)),
            pl.BlockSpec((1, NHASH, S, 1), lambda i: (i, 0, 0, 0)),
            pl.BlockSpec((1, NHASH, 1, 2 * NB), lambda i: (i, 0, 0, 0)),
        ],
        out_shape=[
            jax.ShapeDtypeStruct((B * H, S, 2 * DH), _f32),
            jax.ShapeDtypeStruct((B * H, NHASH, S, 1), jnp.int32),
            jax.ShapeDtypeStruct((B * H, NHASH, 1, 2 * NB), _f32),
        ],
        compiler_params=pltpu.CompilerParams(
            dimension_semantics=("arbitrary",)),
    )(qkh, vh, rot_i)


def _sc_sort(vin_flat, gidx):
    @functools.partial(
        pl.kernel,
        mesh=plsc.VectorSubcoreMesh(core_axis_name="c", subcore_axis_name="s"),
        out_type=jax.ShapeDtypeStruct((GROWS, 2 * DH), _f32),
        scratch_types=[
            pltpu.VMEM((_CHUNK,), jnp.int32),
            pltpu.VMEM((_CHUNK, 2 * DH), _f32),
            pltpu.SemaphoreType.DMA,
        ],
    )
    def k(vin_hbm, idx_hbm, out_hbm, idx_v, rows_v, sem):
        wid = lax.axis_index("s") * _NC + lax.axis_index("c")

        def step(j, carry):
            g0 = wid * _RPW + j * _CHUNK
            t = g0 // S
            src0 = (t // NHASH) * S + (g0 - t * S)
            pltpu.sync_copy(vin_hbm.at[pl.ds(src0, _CHUNK)], rows_v)
            pltpu.sync_copy(idx_hbm.at[pl.ds(g0, _CHUNK)], idx_v)
            pltpu.async_copy(rows_v, out_hbm.at[idx_v], sem).wait()
            return carry

        lax.fori_loop(0, _NSTEP, step, 0)

    return k(vin_flat, gidx)


def _sc_unsort(so_flat, gidx):
    @functools.partial(
        pl.kernel,
        mesh=plsc.VectorSubcoreMesh(core_axis_name="c", subcore_axis_name="s"),
        out_type=jax.ShapeDtypeStruct((GROWS, CH), _f32),
        scratch_types=[
            pltpu.VMEM((_CHUNK,), jnp.int32),
            pltpu.VMEM((_CHUNK, CH), _f32),
            pltpu.SemaphoreType.DMA,
        ],
    )
    def k(so_hbm, idx_hbm, out_hbm, idx_v, rows_v, sem):
        wid = lax.axis_index("s") * _NC + lax.axis_index("c")

        def step(j, carry):
            g0 = wid * _RPW + j * _CHUNK
            pltpu.sync_copy(idx_hbm.at[pl.ds(g0, _CHUNK)], idx_v)
            pltpu.async_copy(so_hbm.at[idx_v], rows_v, sem).wait()
            pltpu.sync_copy(rows_v, out_hbm.at[pl.ds(g0, _CHUNK)])
            return carry

        lax.fori_loop(0, _NSTEP, step, 0)

    return k(so_flat, gidx)


def _attn2_body(svin_ref, oe_ref, so_ref):
    sqk = svin_ref[0, :, 0:DH]
    sv = svin_ref[0, :, DH:2 * DH]
    offsets = oe_ref[0, :, 0:NB]
    ends = oe_ref[0, :, NB:2 * NB]

    riota = _fiota((S, 1), 0)
    son = jnp.where((riota >= offsets) & (riota < ends), 1.0, 0.0)

    kn = sqk * (lax.reciprocal(
        jnp.sqrt(jnp.sum(sqk * sqk, axis=1, keepdims=True)) + 1e-6)
        * (1.0 / np.sqrt(DH)))

    si = lax.broadcasted_iota(jnp.int32, (QR, KR), 0)
    li = lax.broadcasted_iota(jnp.int32, (QR, KR), 1)
    cbase = (si // BUCKET) * BUCKET
    band = (li >= cbase) & (li < cbase + 2 * BUCKET)
    selfm = li == si + BUCKET
    stat = jnp.where(selfm, -1e5, jnp.where(band, 0.0, -1e9))
    pad = jnp.zeros((QR, CH - DH - 1), _f32)

    def kseg(arr, g):
        if g == 0:
            return jnp.concatenate([arr[S - BUCKET:S], arr[0:QR]], axis=0)
        return arr[g * QR - BUCKET:g * QR + QR]

    outs = []
    for g in range(NG):
        q = sqk[g * QR:(g + 1) * QR]
        dots = _dot(q, kseg(kn, g), dn=((1,), (1,)))
        eq = _dot(son[g * QR:(g + 1) * QR], kseg(son, g),
                  dn=((1,), (1,)))
        dots = jnp.where(eq > 0.5, dots, -1e9) + stat
        m = jnp.max(dots, axis=1, keepdims=True)
        e = jnp.exp(dots - m)
        se = jnp.sum(e, axis=1, keepdims=True)
        lse = m + jnp.log(se)
        probs = e * lax.reciprocal(se)
        co = _dot(probs, kseg(sv, g))
        outs.append(jnp.concatenate([co, lse, pad], axis=1))
    so_ref[0] = jnp.concatenate(outs, axis=0)


def _attn2(svin, oe2):
    return pl.pallas_call(
        _attn2_body,
        grid=(NTASK,),
        in_specs=[
            pl.BlockSpec((1, S, 2 * DH), lambda i: (i, 0, 0)),
            pl.BlockSpec((1, 1, 2 * NB), lambda i: (i, 0, 0)),
        ],
        out_specs=pl.BlockSpec((1, S, CH), lambda i: (i, 0, 0)),
        out_shape=jax.ShapeDtypeStruct((NTASK, S, CH), _f32),
        compiler_params=pltpu.CompilerParams(
            dimension_semantics=("arbitrary",)),
    )(svin, oe2)


def _comb_body(ouns_ref, ah_ref):
    o0 = ouns_ref[0, 0:S, 0:DH]
    l0 = ouns_ref[0, 0:S, DH:DH + 1]
    o1 = ouns_ref[0, S:2 * S, 0:DH]
    l1 = ouns_ref[0, S:2 * S, DH:DH + 1]
    mm = jnp.maximum(l0, l1)
    e0 = jnp.exp(l0 - mm)
    e1 = jnp.exp(l1 - mm)
    ah_ref[0, 0] = (o0 * e0 + o1 * e1) * lax.reciprocal(e0 + e1)


def _comb(ouns2):
    return pl.pallas_call(
        _comb_body,
        grid=(B * H,),
        in_specs=[pl.BlockSpec((1, NHASH * S, CH), lambda i: (i, 0, 0))],
        out_specs=pl.BlockSpec((1, 1, S, DH),
                               lambda i: (i // H, i % H, 0, 0)),
        out_shape=jax.ShapeDtypeStruct((B, H, S, DH), _f32),
        compiler_params=pltpu.CompilerParams(
            dimension_semantics=("arbitrary",)),
    )(ouns2)


def kernel(X, missing_mask, W_emb, b_emb, Wqk, Wv, Wo, ln1_g, ln1_b,
           W1, b1, W2, b2, ln2_g, ln2_b, W_out, b_out):
    X2 = X.reshape(RT, F)
    M2 = missing_mask.reshape(RT, F)

    emb = _row_tiled(_emb_body,
                     (('t', F), ('t', F), (F, D), (F, D), (1, D)), (D,))
    x = emb(X2, M2, W_emb[:F], W_emb[F:], b_emb.reshape(1, D))

    rot = jax.random.normal(jax.random.key(42), (L, DH, NHASH, NB // 2), _f32)

    hspec = pl.BlockSpec((1, H, ROWT, DH), lambda i: (i // NST, 0, i % NST, 0))
    qkv = pl.pallas_call(
        _qkv_body,
        grid=(NRT,),
        in_specs=[
            pl.BlockSpec((ROWT, D), lambda i: (i, 0)),
            pl.BlockSpec((D, D), lambda i: (0, 0)),
            pl.BlockSpec((D, D), lambda i: (0, 0)),
        ],
        out_specs=[hspec, hspec],
        out_shape=[jax.ShapeDtypeStruct((B, H, S, DH), _f32)] * 2,
        compiler_params=pltpu.CompilerParams(
            dimension_semantics=("arbitrary",)),
    )
    blk = pl.pallas_call(
        _block_body,
        grid=(NRT,),
        in_specs=[hspec,
                  pl.BlockSpec((ROWT, D), lambda i: (i, 0)),
                  pl.BlockSpec((D, D), lambda i: (0, 0)),
                  pl.BlockSpec((1, D), lambda i: (0, 0)),
                  pl.BlockSpec((1, D), lambda i: (0, 0)),
                  pl.BlockSpec((D, DFF), lambda i: (0, 0)),
                  pl.BlockSpec((1, DFF), lambda i: (0, 0)),
                  pl.BlockSpec((DFF, D), lambda i: (0, 0)),
                  pl.BlockSpec((1, D), lambda i: (0, 0)),
                  pl.BlockSpec((1, D), lambda i: (0, 0)),
                  pl.BlockSpec((1, D), lambda i: (0, 0))],
        out_specs=pl.BlockSpec((ROWT, D), lambda i: (i, 0)),
        out_shape=jax.ShapeDtypeStruct((RT, D), _f32),
        compiler_params=pltpu.CompilerParams(
            dimension_semantics=("arbitrary",)),
    )

    for i in range(L):
        qkh4, vh4 = qkv(x, Wqk[i], Wv[i])
        rot_i = rot[i].transpose(1, 0, 2)

        vin, guns4, oe4 = _prep(qkh4, vh4, rot_i)
        guns = guns4.reshape(GROWS)
        svin = _sc_sort(vin.reshape(VROWS, 2 * DH), guns)
        so = _attn2(svin.reshape(NTASK, S, 2 * DH),
                    oe4.reshape(NTASK, 1, 2 * NB))
        ouns = _sc_unsort(so.reshape(GROWS, CH), guns)
        ah4 = _comb(ouns.reshape(B * H, NHASH * S, CH))

        x = blk(ah4, x, Wo[i], ln1_g[i].reshape(1, D), ln1_b[i].reshape(1, D),
                W1[i], b1[i].reshape(1, DFF), W2[i], b2[i].reshape(1, D),
                ln2_g[i].reshape(1, D), ln2_b[i].reshape(1, D))

    fin = pl.pallas_call(
        _final_body,
        grid=(NRT,),
        in_specs=[
            pl.BlockSpec((ROWT, D), lambda i: (i, 0)),
            pl.BlockSpec((D, F), lambda i: (0, 0)),
            pl.BlockSpec((1, F), lambda i: (0, 0)),
            pl.BlockSpec((ROWT, F), lambda i: (i, 0)),
            pl.BlockSpec((ROWT, F), lambda i: (i, 0)),
        ],
        out_specs=pl.BlockSpec((ROWT, F), lambda i: (i, 0)),
        out_shape=jax.ShapeDtypeStruct((RT, F), _f32),
        compiler_params=pltpu.CompilerParams(
            dimension_semantics=("arbitrary",)),
    )
    out = fin(x, W_out, b_out.reshape(1, F), X2, M2)
    return out.reshape(B, S, F)


# speedup vs baseline: 4.9401x; 1.0424x over previous
import functools

import jax
import jax.numpy as jnp
import numpy as np
from jax import lax
from jax.experimental import pallas as pl
from jax.experimental.pallas import tpu as pltpu
from jax.experimental.pallas import tpu_sc as plsc

B, S, F, D, H, L = 2, 2048, 128, 1024, 16, 2
DH = D // H
DFF = 2048
BUCKET = 64
NHASH = 2
NB = S // BUCKET
RT = B * S
ROWT = 512
NRT = RT // ROWT
NST = S // ROWT
CH = 128
NCB = S // CH
GC = 4
QR = GC * BUCKET
KR = QR + BUCKET
NG = S // QR
NTASK = B * H * NHASH
VROWS = B * H * S
GROWS = NTASK * S

_NC, _NS = 2, 16
_NW = _NC * _NS
_CHUNK = 128
_RPW = GROWS // _NW
_NSTEP = _RPW // _CHUNK

_f32 = jnp.float32


def _fiota(shape, dim):
    return lax.broadcasted_iota(jnp.int32, shape, dim).astype(_f32)


def _ln(x, g, b):
    m = jnp.mean(x, axis=1, keepdims=True)
    v = jnp.mean((x - m) ** 2, axis=1, keepdims=True)
    return (x - m) * lax.rsqrt(v + 1e-5) * g + b


def _dot(a, b, dn=None):
    if dn is None:
        return lax.dot(a, b, preferred_element_type=_f32)
    return lax.dot_general(a, b, dimension_numbers=(dn, ((), ())),
                           preferred_element_type=_f32)


def _emb_body(x_ref, m_ref, wx_ref, wm_ref, b_ref, o_ref):
    o_ref[...] = (_dot(x_ref[...], wx_ref[...]) + _dot(m_ref[...], wm_ref[...])
                  + b_ref[...])


def _qkv_body(x_ref, wqk_ref, wv_ref, qk_ref, v_ref):
    x = x_ref[...]
    qk = _dot(x, wqk_ref[...])
    v = _dot(x, wv_ref[...])
    for h in range(H):
        qk_ref[0, h] = qk[:, h * DH:(h + 1) * DH]
        v_ref[0, h] = v[:, h * DH:(h + 1) * DH]


def _block_body(a_ref, x_ref, wo_ref, g1_ref, b1g_ref, w1_ref, b1_ref,
                w2_ref, b2_ref, g2_ref, b2g_ref, o_ref):
    a = jnp.concatenate([a_ref[0, h] for h in range(H)], axis=1)
    t = x_ref[...] + _dot(a, wo_ref[...])
    t = _ln(t, g1_ref[...], b1g_ref[...])
    ff = jnp.maximum(_dot(t, w1_ref[...]) + b1_ref[...], 0.0)
    ff = _dot(ff, w2_ref[...]) + b2_ref[...]
    o_ref[...] = _ln(t + ff, g2_ref[...], b2g_ref[...])


def _final_body(x_ref, w_ref, b_ref, X_ref, m_ref, o_ref):
    rec = _dot(x_ref[...], w_ref[...]) + b_ref[...]
    m = m_ref[...]
    o_ref[...] = m * X_ref[...] + (1.0 - m) * rec


def _row_tiled(body, in_widths, out_widths):
    in_specs = []
    for w in in_widths:
        if isinstance(w, tuple) and w[0] == 't':
            in_specs.append(pl.BlockSpec((ROWT, w[1]), lambda i: (i, 0)))
        else:
            in_specs.append(pl.BlockSpec(w, lambda i: (0, 0)))
    out_specs = [pl.BlockSpec((ROWT, w), lambda i: (i, 0)) for w in out_widths]
    out_shape = [jax.ShapeDtypeStruct((RT, w), _f32) for w in out_widths]
    return pl.pallas_call(
        body,
        grid=(NRT,),
        in_specs=in_specs,
        out_specs=out_specs if len(out_widths) > 1 else out_specs[0],
        out_shape=out_shape if len(out_widths) > 1 else out_shape[0],
        compiler_params=pltpu.CompilerParams(
            dimension_semantics=("arbitrary",)),
    )


def _prep_body(qk_ref, v_ref, rot_ref, vin_ref, guns_ref, oe_ref):
    pid = pl.program_id(0)
    qk = qk_ref[0, 0]
    v = v_ref[0, 0]
    vin_ref[0] = jnp.concatenate([qk, v], axis=1)

    riota = _fiota((S, 1), 0)
    r128 = _fiota((CH, CH), 0)
    c128 = _fiota((CH, CH), 1)
    Ls = jnp.where(c128 < r128, 1.0, 0.0)
    r32 = _fiota((NB, NB), 0)
    c32 = _fiota((NB, NB), 1)
    Ts32 = jnp.where(r32 < c32, 1.0, 0.0)

    for r in range(NHASH):
        rotated = _dot(qk, rot_ref[r])
        full = jnp.concatenate([rotated, -rotated], axis=1)
        mx = jnp.max(full, axis=1, keepdims=True)
        iota_b = _fiota((S, NB), 1)
        bucket = jnp.min(jnp.where(full >= mx, iota_b, 1e9), axis=1,
                         keepdims=True)
        onehot = jnp.where(iota_b == bucket, 1.0, 0.0)
        blkall = jnp.concatenate(
            [onehot[c * CH:(c + 1) * CH] for c in range(NCB)], axis=1)
        exall = _dot(Ls, blkall)
        lastrow = exall[CH - 1:CH, :] + blkall[CH - 1:CH, :]
        wbps = []
        run = jnp.zeros((1, NB), _f32)
        for c in range(NCB):
            sl = slice(c * NB, (c + 1) * NB)
            wbps.append(exall[:, sl] + run)
            run = run + lastrow[:, sl]
        wbp = jnp.concatenate(wbps, axis=0)
        totals = run
        offsets = _dot(totals, Ts32)
        rank = jnp.sum(onehot * (wbp + offsets), axis=1, keepdims=True)

        base = (pid * NHASH + r) * S
        guns_ref[0, r] = (rank + base.astype(_f32)).astype(jnp.int32)
        oe_ref[0, r] = jnp.concatenate([offsets, offsets + totals], axis=1)


def _prep(qkh, vh, rot_i):
    return pl.pallas_call(
        _prep_body,
        grid=(B * H,),
        in_specs=[
            pl.BlockSpec((1, 1, S, DH), lambda i: (i // H, i % H, 0, 0)),
            pl.BlockSpec((1, 1, S, DH), lambda i: (i // H, i % H, 0, 0)),
            pl.BlockSpec((NHASH, DH, NB // 2), lambda i: (0, 0, 0)),
        ],
        out_specs=[
            pl.BlockSpec((1, S, 2 * DH), lambda i: (i, 0, 0)),
            pl.BlockSpec((1, NHASH, S, 1), lambda i: (i, 0, 0, 0)),
            pl.BlockSpec((1, NHASH, 1, 2 * NB), lambda i: (i, 0, 0, 0)),
        ],
        out_shape=[
            jax.ShapeDtypeStruct((B * H, S, 2 * DH), _f32),
            jax.ShapeDtypeStruct((B * H, NHASH, S, 1), jnp.int32),
            jax.ShapeDtypeStruct((B * H, NHASH, 1, 2 * NB), _f32),
        ],
        compiler_params=pltpu.CompilerParams(
            dimension_semantics=("arbitrary",)),
    )(qkh, vh, rot_i)


def _sc_sort(vin_flat, gidx):
    @functools.partial(
        pl.kernel,
        mesh=plsc.VectorSubcoreMesh(core_axis_name="c", subcore_axis_name="s"),
        out_type=jax.ShapeDtypeStruct((GROWS, 2 * DH), _f32),
        scratch_types=[
            pltpu.VMEM((2, _CHUNK), jnp.int32),
            pltpu.VMEM((2, _CHUNK, 2 * DH), _f32),
            pltpu.SemaphoreType.DMA,
        ],
    )
    def k(vin_hbm, idx_hbm, out_hbm, idx_v, rows_v, sem):
        wid = lax.axis_index("s") * _NC + lax.axis_index("c")

        def step(p, carry):
            cps = []
            for u in range(2):
                g0 = wid * _RPW + (2 * p + u) * _CHUNK
                t = g0 // S
                src0 = (t // NHASH) * S + (g0 - t * S)
                pltpu.sync_copy(vin_hbm.at[pl.ds(src0, _CHUNK)], rows_v.at[u])
                pltpu.sync_copy(idx_hbm.at[pl.ds(g0, _CHUNK)], idx_v.at[u])
                cps.append(pltpu.async_copy(rows_v.at[u],
                                            out_hbm.at[idx_v.at[u]], sem))
            for cp in cps:
                cp.wait()
            return carry

        lax.fori_loop(0, _NSTEP // 2, step, 0)

    return k(vin_flat, gidx)


def _sc_unsort(so_flat, gidx):
    @functools.partial(
        pl.kernel,
        mesh=plsc.VectorSubcoreMesh(core_axis_name="c", subcore_axis_name="s"),
        out_type=jax.ShapeDtypeStruct((GROWS, CH), _f32),
        scratch_types=[
            pltpu.VMEM((2, _CHUNK), jnp.int32),
            pltpu.VMEM((2, _CHUNK, CH), _f32),
            pltpu.SemaphoreType.DMA,
        ],
    )
    def k(so_hbm, idx_hbm, out_hbm, idx_v, rows_v, sem):
        wid = lax.axis_index("s") * _NC + lax.axis_index("c")

        def step(p, carry):
            cps = []
            for u in range(2):
                g0 = wid * _RPW + (2 * p + u) * _CHUNK
                pltpu.sync_copy(idx_hbm.at[pl.ds(g0, _CHUNK)], idx_v.at[u])
                cps.append(pltpu.async_copy(so_hbm.at[idx_v.at[u]],
                                            rows_v.at[u], sem))
            for u in range(2):
                g0 = wid * _RPW + (2 * p + u) * _CHUNK
                cps[u].wait()
                pltpu.sync_copy(rows_v.at[u], out_hbm.at[pl.ds(g0, _CHUNK)])
            return carry

        lax.fori_loop(0, _NSTEP // 2, step, 0)

    return k(so_flat, gidx)


def _attn2_body(svin_ref, oe_ref, so_ref):
    sqk = svin_ref[0, :, 0:DH]
    sv = svin_ref[0, :, DH:2 * DH]
    offsets = oe_ref[0, :, 0:NB]
    ends = oe_ref[0, :, NB:2 * NB]

    riota = _fiota((S, 1), 0)
    son = jnp.where((riota >= offsets) & (riota < ends), 1.0, 0.0)

    kn = sqk * (lax.reciprocal(
        jnp.sqrt(jnp.sum(sqk * sqk, axis=1, keepdims=True)) + 1e-6)
        * (1.0 / np.sqrt(DH)))

    si = lax.broadcasted_iota(jnp.int32, (QR, KR), 0)
    li = lax.broadcasted_iota(jnp.int32, (QR, KR), 1)
    cbase = (si // BUCKET) * BUCKET
    band = (li >= cbase) & (li < cbase + 2 * BUCKET)
    selfm = li == si + BUCKET
    stat = jnp.where(selfm, -1e5, jnp.where(band, 0.0, -1e9))
    pad = jnp.zeros((QR, CH - DH - 1), _f32)

    def kseg(arr, g):
        if g == 0:
            return jnp.concatenate([arr[S - BUCKET:S], arr[0:QR]], axis=0)
        return arr[g * QR - BUCKET:g * QR + QR]

    outs = []
    for g in range(NG):
        q = sqk[g * QR:(g + 1) * QR]
        dots = _dot(q, kseg(kn, g), dn=((1,), (1,)))
        eq = _dot(son[g * QR:(g + 1) * QR], kseg(son, g),
                  dn=((1,), (1,)))
        dots = jnp.where(eq > 0.5, dots, -1e9) + stat
        m = jnp.max(dots, axis=1, keepdims=True)
        e = jnp.exp(dots - m)
        se = jnp.sum(e, axis=1, keepdims=True)
        lse = m + jnp.log(se)
        probs = e * lax.reciprocal(se)
        co = _dot(probs, kseg(sv, g))
        outs.append(jnp.concatenate([co, lse, pad], axis=1))
    so_ref[0] = jnp.concatenate(outs, axis=0)


def _attn2(svin, oe2):
    return pl.pallas_call(
        _attn2_body,
        grid=(NTASK,),
        in_specs=[
            pl.BlockSpec((1, S, 2 * DH), lambda i: (i, 0, 0)),
            pl.BlockSpec((1, 1, 2 * NB), lambda i: (i, 0, 0)),
        ],
        out_specs=pl.BlockSpec((1, S, CH), lambda i: (i, 0, 0)),
        out_shape=jax.ShapeDtypeStruct((NTASK, S, CH), _f32),
        compiler_params=pltpu.CompilerParams(
            dimension_semantics=("arbitrary",)),
    )(svin, oe2)


def _comb_body(ouns_ref, ah_ref):
    o0 = ouns_ref[0, 0:S, 0:DH]
    l0 = ouns_ref[0, 0:S, DH:DH + 1]
    o1 = ouns_ref[0, S:2 * S, 0:DH]
    l1 = ouns_ref[0, S:2 * S, DH:DH + 1]
    mm = jnp.maximum(l0, l1)
    e0 = jnp.exp(l0 - mm)
    e1 = jnp.exp(l1 - mm)
    ah_ref[0, 0] = (o0 * e0 + o1 * e1) * lax.reciprocal(e0 + e1)


def _comb(ouns2):
    return pl.pallas_call(
        _comb_body,
        grid=(B * H,),
        in_specs=[pl.BlockSpec((1, NHASH * S, CH), lambda i: (i, 0, 0))],
        out_specs=pl.BlockSpec((1, 1, S, DH),
                               lambda i: (i // H, i % H, 0, 0)),
        out_shape=jax.ShapeDtypeStruct((B, H, S, DH), _f32),
        compiler_params=pltpu.CompilerParams(
            dimension_semantics=("arbitrary",)),
    )(ouns2)


def kernel(X, missing_mask, W_emb, b_emb, Wqk, Wv, Wo, ln1_g, ln1_b,
           W1, b1, W2, b2, ln2_g, ln2_b, W_out, b_out):
    X2 = X.reshape(RT, F)
    M2 = missing_mask.reshape(RT, F)

    emb = _row_tiled(_emb_body,
                     (('t', F), ('t', F), (F, D), (F, D), (1, D)), (D,))
    x = emb(X2, M2, W_emb[:F], W_emb[F:], b_emb.reshape(1, D))

    rot = jax.random.normal(jax.random.key(42), (L, DH, NHASH, NB // 2), _f32)

    hspec = pl.BlockSpec((1, H, ROWT, DH), lambda i: (i // NST, 0, i % NST, 0))
    qkv = pl.pallas_call(
        _qkv_body,
        grid=(NRT,),
        in_specs=[
            pl.BlockSpec((ROWT, D), lambda i: (i, 0)),
            pl.BlockSpec((D, D), lambda i: (0, 0)),
            pl.BlockSpec((D, D), lambda i: (0, 0)),
        ],
        out_specs=[hspec, hspec],
        out_shape=[jax.ShapeDtypeStruct((B, H, S, DH), _f32)] * 2,
        compiler_params=pltpu.CompilerParams(
            dimension_semantics=("arbitrary",)),
    )
    blk = pl.pallas_call(
        _block_body,
        grid=(NRT,),
        in_specs=[hspec,
                  pl.BlockSpec((ROWT, D), lambda i: (i, 0)),
                  pl.BlockSpec((D, D), lambda i: (0, 0)),
                  pl.BlockSpec((1, D), lambda i: (0, 0)),
                  pl.BlockSpec((1, D), lambda i: (0, 0)),
                  pl.BlockSpec((D, DFF), lambda i: (0, 0)),
                  pl.BlockSpec((1, DFF), lambda i: (0, 0)),
                  pl.BlockSpec((DFF, D), lambda i: (0, 0)),
                  pl.BlockSpec((1, D), lambda i: (0, 0)),
                  pl.BlockSpec((1, D), lambda i: (0, 0)),
                  pl.BlockSpec((1, D), lambda i: (0, 0))],
        out_specs=pl.BlockSpec((ROWT, D), lambda i: (i, 0)),
        out_shape=jax.ShapeDtypeStruct((RT, D), _f32),
        compiler_params=pltpu.CompilerParams(
            dimension_semantics=("arbitrary",)),
    )

    for i in range(L):
        qkh4, vh4 = qkv(x, Wqk[i], Wv[i])
        rot_i = rot[i].transpose(1, 0, 2)

        vin, guns4, oe4 = _prep(qkh4, vh4, rot_i)
        guns = guns4.reshape(GROWS)
        svin = _sc_sort(vin.reshape(VROWS, 2 * DH), guns)
        so = _attn2(svin.reshape(NTASK, S, 2 * DH),
                    oe4.reshape(NTASK, 1, 2 * NB))
        ouns = _sc_unsort(so.reshape(GROWS, CH), guns)
        ah4 = _comb(ouns.reshape(B * H, NHASH * S, CH))

        x = blk(ah4, x, Wo[i], ln1_g[i].reshape(1, D), ln1_b[i].reshape(1, D),
                W1[i], b1[i].reshape(1, DFF), W2[i], b2[i].reshape(1, D),
                ln2_g[i].reshape(1, D), ln2_b[i].reshape(1, D))

    fin = pl.pallas_call(
        _final_body,
        grid=(NRT,),
        in_specs=[
            pl.BlockSpec((ROWT, D), lambda i: (i, 0)),
            pl.BlockSpec((D, F), lambda i: (0, 0)),
            pl.BlockSpec((1, F), lambda i: (0, 0)),
            pl.BlockSpec((ROWT, F), lambda i: (i, 0)),
            pl.BlockSpec((ROWT, F), lambda i: (i, 0)),
        ],
        out_specs=pl.BlockSpec((ROWT, F), lambda i: (i, 0)),
        out_shape=jax.ShapeDtypeStruct((RT, F), _f32),
        compiler_params=pltpu.CompilerParams(
            dimension_semantics=("arbitrary",)),
    )
    out = fin(x, W_out, b_out.reshape(1, F), X2, M2)
    return out.reshape(B, S, F)


# speedup vs baseline: 4.9985x; 1.0118x over previous
import functools

import jax
import jax.numpy as jnp
import numpy as np
from jax import lax
from jax.experimental import pallas as pl
from jax.experimental.pallas import tpu as pltpu
from jax.experimental.pallas import tpu_sc as plsc

B, S, F, D, H, L = 2, 2048, 128, 1024, 16, 2
DH = D // H
DFF = 2048
BUCKET = 64
NHASH = 2
NB = S // BUCKET
RT = B * S
ROWT = 512
NRT = RT // ROWT
NST = S // ROWT
CH = 128
NCB = S // CH
GC = 4
QR = GC * BUCKET
KR = QR + BUCKET
NG = S // QR
NTASK = B * H * NHASH
VROWS = B * H * S
GROWS = NTASK * S

_NC, _NS = 2, 16
_NW = _NC * _NS
_CHUNK = 128
_RPW = GROWS // _NW
_NSTEP = _RPW // _CHUNK

_f32 = jnp.float32


def _fiota(shape, dim):
    return lax.broadcasted_iota(jnp.int32, shape, dim).astype(_f32)


def _ln(x, g, b):
    m = jnp.mean(x, axis=1, keepdims=True)
    v = jnp.mean((x - m) ** 2, axis=1, keepdims=True)
    return (x - m) * lax.rsqrt(v + 1e-5) * g + b


def _dot(a, b, dn=None):
    if dn is None:
        return lax.dot(a, b, preferred_element_type=_f32)
    return lax.dot_general(a, b, dimension_numbers=(dn, ((), ())),
                           preferred_element_type=_f32)


def _emb_body(x_ref, m_ref, wx_ref, wm_ref, b_ref, o_ref):
    o_ref[...] = (_dot(x_ref[...], wx_ref[...]) + _dot(m_ref[...], wm_ref[...])
                  + b_ref[...])


def _qkv_body(x_ref, wqk_ref, wv_ref, qk_ref, v_ref):
    x = x_ref[...]
    qk = _dot(x, wqk_ref[...])
    v = _dot(x, wv_ref[...])
    for h in range(H):
        qk_ref[0, h] = qk[:, h * DH:(h + 1) * DH]
        v_ref[0, h] = v[:, h * DH:(h + 1) * DH]


def _block_body(a_ref, x_ref, wo_ref, g1_ref, b1g_ref, w1_ref, b1_ref,
                w2_ref, b2_ref, g2_ref, b2g_ref, o_ref):
    a = jnp.concatenate([a_ref[0, h] for h in range(H)], axis=1)
    t = x_ref[...] + _dot(a, wo_ref[...])
    t = _ln(t, g1_ref[...], b1g_ref[...])
    ff = jnp.maximum(_dot(t, w1_ref[...]) + b1_ref[...], 0.0)
    ff = _dot(ff, w2_ref[...]) + b2_ref[...]
    o_ref[...] = _ln(t + ff, g2_ref[...], b2g_ref[...])


def _final_body(x_ref, w_ref, b_ref, X_ref, m_ref, o_ref):
    rec = _dot(x_ref[...], w_ref[...]) + b_ref[...]
    m = m_ref[...]
    o_ref[...] = m * X_ref[...] + (1.0 - m) * rec


def _row_tiled(body, in_widths, out_widths):
    in_specs = []
    for w in in_widths:
        if isinstance(w, tuple) and w[0] == 't':
            in_specs.append(pl.BlockSpec((ROWT, w[1]), lambda i: (i, 0)))
        else:
            in_specs.append(pl.BlockSpec(w, lambda i: (0, 0)))
    out_specs = [pl.BlockSpec((ROWT, w), lambda i: (i, 0)) for w in out_widths]
    out_shape = [jax.ShapeDtypeStruct((RT, w), _f32) for w in out_widths]
    return pl.pallas_call(
        body,
        grid=(NRT,),
        in_specs=in_specs,
        out_specs=out_specs if len(out_widths) > 1 else out_specs[0],
        out_shape=out_shape if len(out_widths) > 1 else out_shape[0],
        compiler_params=pltpu.CompilerParams(
            dimension_semantics=("arbitrary",)),
    )


def _prep_body(qk_ref, v_ref, rot_ref, vin_ref, guns_ref, oe_ref):
    pid = pl.program_id(0)
    qk = qk_ref[0, 0]
    v = v_ref[0, 0]
    vin_ref[0] = jnp.concatenate([qk, v], axis=1)

    riota = _fiota((S, 1), 0)
    r128 = _fiota((CH, CH), 0)
    c128 = _fiota((CH, CH), 1)
    Ls = jnp.where(c128 < r128, 1.0, 0.0)
    r32 = _fiota((NB, NB), 0)
    c32 = _fiota((NB, NB), 1)
    Ts32 = jnp.where(r32 < c32, 1.0, 0.0)

    for r in range(NHASH):
        rotated = _dot(qk, rot_ref[r])
        full = jnp.concatenate([rotated, -rotated], axis=1)
        mx = jnp.max(full, axis=1, keepdims=True)
        iota_b = _fiota((S, NB), 1)
        bucket = jnp.min(jnp.where(full >= mx, iota_b, 1e9), axis=1,
                         keepdims=True)
        onehot = jnp.where(iota_b == bucket, 1.0, 0.0)
        blkall = jnp.concatenate(
            [onehot[c * CH:(c + 1) * CH] for c in range(NCB)], axis=1)
        exall = _dot(Ls, blkall)
        lastrow = exall[CH - 1:CH, :] + blkall[CH - 1:CH, :]
        wbps = []
        run = jnp.zeros((1, NB), _f32)
        for c in range(NCB):
            sl = slice(c * NB, (c + 1) * NB)
            wbps.append(exall[:, sl] + run)
            run = run + lastrow[:, sl]
        wbp = jnp.concatenate(wbps, axis=0)
        totals = run
        offsets = _dot(totals, Ts32)
        rank = jnp.sum(onehot * (wbp + offsets), axis=1, keepdims=True)

        base = (pid * NHASH + r) * S
        guns_ref[0, r] = (rank + base.astype(_f32)).astype(jnp.int32)
        oe_ref[0, r] = jnp.concatenate([offsets, offsets + totals], axis=1)


def _prep(qkh, vh, rot_i):
    return pl.pallas_call(
        _prep_body,
        grid=(B * H,),
        in_specs=[
            pl.BlockSpec((1, 1, S, DH), lambda i: (i // H, i % H, 0, 0)),
            pl.BlockSpec((1, 1, S, DH), lambda i: (i // H, i % H, 0, 0)),
            pl.BlockSpec((NHASH, DH, NB // 2), lambda i: (0, 0, 0)),
        ],
        out_specs=[
            pl.BlockSpec((1, S, 2 * DH), lambda i: (i, 0, 0)),
            pl.BlockSpec((1, NHASH, S, 1), lambda i: (i, 0, 0, 0)),
            pl.BlockSpec((1, NHASH, 1, 2 * NB), lambda i: (i, 0, 0, 0)),
        ],
        out_shape=[
            jax.ShapeDtypeStruct((B * H, S, 2 * DH), _f32),
            jax.ShapeDtypeStruct((B * H, NHASH, S, 1), jnp.int32),
            jax.ShapeDtypeStruct((B * H, NHASH, 1, 2 * NB), _f32),
        ],
        compiler_params=pltpu.CompilerParams(
            dimension_semantics=("arbitrary",)),
    )(qkh, vh, rot_i)


def _sc_sort(vin_flat, gidx):
    @functools.partial(
        pl.kernel,
        mesh=plsc.VectorSubcoreMesh(core_axis_name="c", subcore_axis_name="s"),
        out_type=jax.ShapeDtypeStruct((GROWS, 2 * DH), _f32),
        scratch_types=[
            pltpu.VMEM((4, _CHUNK), jnp.int32),
            pltpu.VMEM((4, _CHUNK, 2 * DH), _f32),
            pltpu.SemaphoreType.DMA,
        ],
    )
    def k(vin_hbm, idx_hbm, out_hbm, idx_v, rows_v, sem):
        wid = lax.axis_index("s") * _NC + lax.axis_index("c")

        def step(p, carry):
            cps = []
            for u in range(4):
                g0 = wid * _RPW + (4 * p + u) * _CHUNK
                t = g0 // S
                src0 = (t // NHASH) * S + (g0 - t * S)
                pltpu.sync_copy(vin_hbm.at[pl.ds(src0, _CHUNK)], rows_v.at[u])
                pltpu.sync_copy(idx_hbm.at[pl.ds(g0, _CHUNK)], idx_v.at[u])
                cps.append(pltpu.async_copy(rows_v.at[u],
                                            out_hbm.at[idx_v.at[u]], sem))
            for cp in cps:
                cp.wait()
            return carry

        lax.fori_loop(0, _NSTEP // 4, step, 0)

    return k(vin_flat, gidx)


def _sc_unsort(so_flat, gidx):
    @functools.partial(
        pl.kernel,
        mesh=plsc.VectorSubcoreMesh(core_axis_name="c", subcore_axis_name="s"),
        out_type=jax.ShapeDtypeStruct((GROWS, CH), _f32),
        scratch_types=[
            pltpu.VMEM((4, _CHUNK), jnp.int32),
            pltpu.VMEM((4, _CHUNK, CH), _f32),
            pltpu.SemaphoreType.DMA,
        ],
    )
    def k(so_hbm, idx_hbm, out_hbm, idx_v, rows_v, sem):
        wid = lax.axis_index("s") * _NC + lax.axis_index("c")

        def step(p, carry):
            cps = []
            for u in range(4):
                g0 = wid * _RPW + (4 * p + u) * _CHUNK
                pltpu.sync_copy(idx_hbm.at[pl.ds(g0, _CHUNK)], idx_v.at[u])
                cps.append(pltpu.async_copy(so_hbm.at[idx_v.at[u]],
                                            rows_v.at[u], sem))
            for u in range(4):
                g0 = wid * _RPW + (4 * p + u) * _CHUNK
                cps[u].wait()
                pltpu.sync_copy(rows_v.at[u], out_hbm.at[pl.ds(g0, _CHUNK)])
            return carry

        lax.fori_loop(0, _NSTEP // 4, step, 0)

    return k(so_flat, gidx)


def _attn2_body(svin_ref, oe_ref, so_ref):
    sqk = svin_ref[0, :, 0:DH]
    sv = svin_ref[0, :, DH:2 * DH]
    offsets = oe_ref[0, :, 0:NB]
    ends = oe_ref[0, :, NB:2 * NB]

    riota = _fiota((S, 1), 0)
    son = jnp.where((riota >= offsets) & (riota < ends), 1.0, 0.0)

    kn = sqk * (lax.reciprocal(
        jnp.sqrt(jnp.sum(sqk * sqk, axis=1, keepdims=True)) + 1e-6)
        * (1.0 / np.sqrt(DH)))

    si = lax.broadcasted_iota(jnp.int32, (QR, KR), 0)
    li = lax.broadcasted_iota(jnp.int32, (QR, KR), 1)
    cbase = (si // BUCKET) * BUCKET
    band = (li >= cbase) & (li < cbase + 2 * BUCKET)
    selfm = li == si + BUCKET
    stat = jnp.where(selfm, -1e5, jnp.where(band, 0.0, -1e9))
    pad = jnp.zeros((QR, CH - DH - 1), _f32)

    def kseg(arr, g):
        if g == 0:
            return jnp.concatenate([arr[S - BUCKET:S], arr[0:QR]], axis=0)
        return arr[g * QR - BUCKET:g * QR + QR]

    outs = []
    for g in range(NG):
        q = sqk[g * QR:(g + 1) * QR]
        dots = _dot(q, kseg(kn, g), dn=((1,), (1,)))
        eq = _dot(son[g * QR:(g + 1) * QR], kseg(son, g),
                  dn=((1,), (1,)))
        dots = jnp.where(eq > 0.5, dots, -1e9) + stat
        m = jnp.max(dots, axis=1, keepdims=True)
        e = jnp.exp(dots - m)
        se = jnp.sum(e, axis=1, keepdims=True)
        lse = m + jnp.log(se)
        probs = e * lax.reciprocal(se)
        co = _dot(probs, kseg(sv, g))
        outs.append(jnp.concatenate([co, lse, pad], axis=1))
    so_ref[0] = jnp.concatenate(outs, axis=0)


def _attn2(svin, oe2):
    return pl.pallas_call(
        _attn2_body,
        grid=(NTASK,),
        in_specs=[
            pl.BlockSpec((1, S, 2 * DH), lambda i: (i, 0, 0)),
            pl.BlockSpec((1, 1, 2 * NB), lambda i: (i, 0, 0)),
        ],
        out_specs=pl.BlockSpec((1, S, CH), lambda i: (i, 0, 0)),
        out_shape=jax.ShapeDtypeStruct((NTASK, S, CH), _f32),
        compiler_params=pltpu.CompilerParams(
            dimension_semantics=("arbitrary",)),
    )(svin, oe2)


def _comb_body(ouns_ref, ah_ref):
    o0 = ouns_ref[0, 0:S, 0:DH]
    l0 = ouns_ref[0, 0:S, DH:DH + 1]
    o1 = ouns_ref[0, S:2 * S, 0:DH]
    l1 = ouns_ref[0, S:2 * S, DH:DH + 1]
    mm = jnp.maximum(l0, l1)
    e0 = jnp.exp(l0 - mm)
    e1 = jnp.exp(l1 - mm)
    ah_ref[0, 0] = (o0 * e0 + o1 * e1) * lax.reciprocal(e0 + e1)


def _comb(ouns2):
    return pl.pallas_call(
        _comb_body,
        grid=(B * H,),
        in_specs=[pl.BlockSpec((1, NHASH * S, CH), lambda i: (i, 0, 0))],
        out_specs=pl.BlockSpec((1, 1, S, DH),
                               lambda i: (i // H, i % H, 0, 0)),
        out_shape=jax.ShapeDtypeStruct((B, H, S, DH), _f32),
        compiler_params=pltpu.CompilerParams(
            dimension_semantics=("arbitrary",)),
    )(ouns2)


def kernel(X, missing_mask, W_emb, b_emb, Wqk, Wv, Wo, ln1_g, ln1_b,
           W1, b1, W2, b2, ln2_g, ln2_b, W_out, b_out):
    X2 = X.reshape(RT, F)
    M2 = missing_mask.reshape(RT, F)

    emb = _row_tiled(_emb_body,
                     (('t', F), ('t', F), (F, D), (F, D), (1, D)), (D,))
    x = emb(X2, M2, W_emb[:F], W_emb[F:], b_emb.reshape(1, D))

    rot = jax.random.normal(jax.random.key(42), (L, DH, NHASH, NB // 2), _f32)

    hspec = pl.BlockSpec((1, H, ROWT, DH), lambda i: (i // NST, 0, i % NST, 0))
    qkv = pl.pallas_call(
        _qkv_body,
        grid=(NRT,),
        in_specs=[
            pl.BlockSpec((ROWT, D), lambda i: (i, 0)),
            pl.BlockSpec((D, D), lambda i: (0, 0)),
            pl.BlockSpec((D, D), lambda i: (0, 0)),
        ],
        out_specs=[hspec, hspec],
        out_shape=[jax.ShapeDtypeStruct((B, H, S, DH), _f32)] * 2,
        compiler_params=pltpu.CompilerParams(
            dimension_semantics=("arbitrary",)),
    )
    blk = pl.pallas_call(
        _block_body,
        grid=(NRT,),
        in_specs=[hspec,
                  pl.BlockSpec((ROWT, D), lambda i: (i, 0)),
                  pl.BlockSpec((D, D), lambda i: (0, 0)),
                  pl.BlockSpec((1, D), lambda i: (0, 0)),
                  pl.BlockSpec((1, D), lambda i: (0, 0)),
                  pl.BlockSpec((D, DFF), lambda i: (0, 0)),
                  pl.BlockSpec((1, DFF), lambda i: (0, 0)),
                  pl.BlockSpec((DFF, D), lambda i: (0, 0)),
                  pl.BlockSpec((1, D), lambda i: (0, 0)),
                  pl.BlockSpec((1, D), lambda i: (0, 0)),
                  pl.BlockSpec((1, D), lambda i: (0, 0))],
        out_specs=pl.BlockSpec((ROWT, D), lambda i: (i, 0)),
        out_shape=jax.ShapeDtypeStruct((RT, D), _f32),
        compiler_params=pltpu.CompilerParams(
            dimension_semantics=("arbitrary",)),
    )

    for i in range(L):
        qkh4, vh4 = qkv(x, Wqk[i], Wv[i])
        rot_i = rot[i].transpose(1, 0, 2)

        vin, guns4, oe4 = _prep(qkh4, vh4, rot_i)
        guns = guns4.reshape(GROWS)
        svin = _sc_sort(vin.reshape(VROWS, 2 * DH), guns)
        so = _attn2(svin.reshape(NTASK, S, 2 * DH),
                    oe4.reshape(NTASK, 1, 2 * NB))
        ouns = _sc_unsort(so.reshape(GROWS, CH), guns)
        ah4 = _comb(ouns.reshape(B * H, NHASH * S, CH))

        x = blk(ah4, x, Wo[i], ln1_g[i].reshape(1, D), ln1_b[i].reshape(1, D),
                W1[i], b1[i].reshape(1, DFF), W2[i], b2[i].reshape(1, D),
                ln2_g[i].reshape(1, D), ln2_b[i].reshape(1, D))

    fin = pl.pallas_call(
        _final_body,
        grid=(NRT,),
        in_specs=[
            pl.BlockSpec((ROWT, D), lambda i: (i, 0)),
            pl.BlockSpec((D, F), lambda i: (0, 0)),
            pl.BlockSpec((1, F), lambda i: (0, 0)),
            pl.BlockSpec((ROWT, F), lambda i: (i, 0)),
            pl.BlockSpec((ROWT, F), lambda i: (i, 0)),
        ],
        out_specs=pl.BlockSpec((ROWT, F), lambda i: (i, 0)),
        out_shape=jax.ShapeDtypeStruct((RT, F), _f32),
        compiler_params=pltpu.CompilerParams(
            dimension_semantics=("arbitrary",)),
    )
    out = fin(x, W_out, b_out.reshape(1, F), X2, M2)
    return out.reshape(B, S, F)


# speedup vs baseline: 5.1575x; 1.0318x over previous
import functools

import jax
import jax.numpy as jnp
import numpy as np
from jax import lax
from jax.experimental import pallas as pl
from jax.experimental.pallas import tpu as pltpu
from jax.experimental.pallas import tpu_sc as plsc

B, S, F, D, H, L = 2, 2048, 128, 1024, 16, 2
DH = D // H
DFF = 2048
BUCKET = 64
NHASH = 2
NB = S // BUCKET
RT = B * S
ROWT = 512
NRT = RT // ROWT
NST = S // ROWT
CH = 128
NCB = S // CH
GC = 4
QR = GC * BUCKET
KR = QR + BUCKET
NG = S // QR
NTASK = B * H * NHASH
VROWS = B * H * S
GROWS = NTASK * S
NTASK_H = H * NHASH
VROWS_H = H * S
GROWS_H = NTASK_H * S

_NC, _NS = 2, 16
_NW = _NC * _NS
_CHUNK = 128
_RPW = GROWS_H // _NW
_NSTEP = _RPW // _CHUNK

_f32 = jnp.float32


def _fiota(shape, dim):
    return lax.broadcasted_iota(jnp.int32, shape, dim).astype(_f32)


def _ln(x, g, b):
    m = jnp.mean(x, axis=1, keepdims=True)
    v = jnp.mean((x - m) ** 2, axis=1, keepdims=True)
    return (x - m) * lax.rsqrt(v + 1e-5) * g + b


def _dot(a, b, dn=None):
    if dn is None:
        return lax.dot(a, b, preferred_element_type=_f32)
    return lax.dot_general(a, b, dimension_numbers=(dn, ((), ())),
                           preferred_element_type=_f32)


def _emb_body(x_ref, m_ref, wx_ref, wm_ref, b_ref, o_ref):
    o_ref[...] = (_dot(x_ref[...], wx_ref[...]) + _dot(m_ref[...], wm_ref[...])
                  + b_ref[...])


def _qkv_body(x_ref, wqk_ref, wv_ref, qk_ref, v_ref):
    x = x_ref[...]
    qk = _dot(x, wqk_ref[...])
    v = _dot(x, wv_ref[...])
    for h in range(H):
        qk_ref[0, h] = qk[:, h * DH:(h + 1) * DH]
        v_ref[0, h] = v[:, h * DH:(h + 1) * DH]


def _block_body(a_ref, x_ref, wo_ref, g1_ref, b1g_ref, w1_ref, b1_ref,
                w2_ref, b2_ref, g2_ref, b2g_ref, o_ref):
    a = jnp.concatenate([a_ref[0, h] for h in range(H)], axis=1)
    t = x_ref[...] + _dot(a, wo_ref[...])
    t = _ln(t, g1_ref[...], b1g_ref[...])
    ff = jnp.maximum(_dot(t, w1_ref[...]) + b1_ref[...], 0.0)
    ff = _dot(ff, w2_ref[...]) + b2_ref[...]
    o_ref[...] = _ln(t + ff, g2_ref[...], b2g_ref[...])


def _final_body(x_ref, w_ref, b_ref, X_ref, m_ref, o_ref):
    rec = _dot(x_ref[...], w_ref[...]) + b_ref[...]
    m = m_ref[...]
    o_ref[...] = m * X_ref[...] + (1.0 - m) * rec


def _row_tiled(body, in_widths, out_widths):
    in_specs = []
    for w in in_widths:
        if isinstance(w, tuple) and w[0] == 't':
            in_specs.append(pl.BlockSpec((ROWT, w[1]), lambda i: (i, 0)))
        else:
            in_specs.append(pl.BlockSpec(w, lambda i: (0, 0)))
    out_specs = [pl.BlockSpec((ROWT, w), lambda i: (i, 0)) for w in out_widths]
    out_shape = [jax.ShapeDtypeStruct((RT, w), _f32) for w in out_widths]
    return pl.pallas_call(
        body,
        grid=(NRT,),
        in_specs=in_specs,
        out_specs=out_specs if len(out_widths) > 1 else out_specs[0],
        out_shape=out_shape if len(out_widths) > 1 else out_shape[0],
        compiler_params=pltpu.CompilerParams(
            dimension_semantics=("arbitrary",)),
    )


def _prep_body(qk_ref, v_ref, rot_ref, vin_ref, guns_ref, oe_ref):
    pid = pl.program_id(0)
    qk = qk_ref[0, 0]
    v = v_ref[0, 0]
    vin_ref[0] = jnp.concatenate([qk, v], axis=1)

    riota = _fiota((S, 1), 0)
    r128 = _fiota((CH, CH), 0)
    c128 = _fiota((CH, CH), 1)
    Ls = jnp.where(c128 < r128, 1.0, 0.0)
    r32 = _fiota((NB, NB), 0)
    c32 = _fiota((NB, NB), 1)
    Ts32 = jnp.where(r32 < c32, 1.0, 0.0)

    for r in range(NHASH):
        rotated = _dot(qk, rot_ref[r])
        full = jnp.concatenate([rotated, -rotated], axis=1)
        mx = jnp.max(full, axis=1, keepdims=True)
        iota_b = _fiota((S, NB), 1)
        bucket = jnp.min(jnp.where(full >= mx, iota_b, 1e9), axis=1,
                         keepdims=True)
        onehot = jnp.where(iota_b == bucket, 1.0, 0.0)
        blkall = jnp.concatenate(
            [onehot[c * CH:(c + 1) * CH] for c in range(NCB)], axis=1)
        exall = _dot(Ls, blkall)
        lastrow = exall[CH - 1:CH, :] + blkall[CH - 1:CH, :]
        wbps = []
        run = jnp.zeros((1, NB), _f32)
        for c in range(NCB):
            sl = slice(c * NB, (c + 1) * NB)
            wbps.append(exall[:, sl] + run)
            run = run + lastrow[:, sl]
        wbp = jnp.concatenate(wbps, axis=0)
        totals = run
        offsets = _dot(totals, Ts32)
        rank = jnp.sum(onehot * (wbp + offsets), axis=1, keepdims=True)

        base = (pid * NHASH + r) * S
        guns_ref[0, r] = (rank + base.astype(_f32)).astype(jnp.int32)
        oe_ref[0, r] = jnp.concatenate([offsets, offsets + totals], axis=1)


def _prep(qkh, vh, rot_i):
    return pl.pallas_call(
        _prep_body,
        grid=(H,),
        in_specs=[
            pl.BlockSpec((1, 1, S, DH), lambda i: (0, i, 0, 0)),
            pl.BlockSpec((1, 1, S, DH), lambda i: (0, i, 0, 0)),
            pl.BlockSpec((NHASH, DH, NB // 2), lambda i: (0, 0, 0)),
        ],
        out_specs=[
            pl.BlockSpec((1, S, 2 * DH), lambda i: (i, 0, 0)),
            pl.BlockSpec((1, NHASH, S, 1), lambda i: (i, 0, 0, 0)),
            pl.BlockSpec((1, NHASH, 1, 2 * NB), lambda i: (i, 0, 0, 0)),
        ],
        out_shape=[
            jax.ShapeDtypeStruct((H, S, 2 * DH), _f32),
            jax.ShapeDtypeStruct((H, NHASH, S, 1), jnp.int32),
            jax.ShapeDtypeStruct((H, NHASH, 1, 2 * NB), _f32),
        ],
        compiler_params=pltpu.CompilerParams(
            dimension_semantics=("arbitrary",)),
    )(qkh, vh, rot_i)


def _sc_sort(vin_flat, gidx):
    @functools.partial(
        pl.kernel,
        mesh=plsc.VectorSubcoreMesh(core_axis_name="c", subcore_axis_name="s"),
        out_type=jax.ShapeDtypeStruct((GROWS_H, 2 * DH), _f32),
        scratch_types=[
            pltpu.VMEM((4, _CHUNK), jnp.int32),
            pltpu.VMEM((4, _CHUNK, 2 * DH), _f32),
            pltpu.SemaphoreType.DMA,
        ],
    )
    def k(vin_hbm, idx_hbm, out_hbm, idx_v, rows_v, sem):
        wid = lax.axis_index("s") * _NC + lax.axis_index("c")

        def step(p, carry):
            cps = []
            for u in range(4):
                g0 = wid * _RPW + (4 * p + u) * _CHUNK
                t = g0 // S
                src0 = (t // NHASH) * S + (g0 - t * S)
                pltpu.sync_copy(vin_hbm.at[pl.ds(src0, _CHUNK)], rows_v.at[u])
                pltpu.sync_copy(idx_hbm.at[pl.ds(g0, _CHUNK)], idx_v.at[u])
                cps.append(pltpu.async_copy(rows_v.at[u],
                                            out_hbm.at[idx_v.at[u]], sem))
            for cp in cps:
                cp.wait()
            return carry

        lax.fori_loop(0, _NSTEP // 4, step, 0)

    return k(vin_flat, gidx)


def _sc_unsort(so_flat, gidx):
    @functools.partial(
        pl.kernel,
        mesh=plsc.VectorSubcoreMesh(core_axis_name="c", subcore_axis_name="s"),
        out_type=jax.ShapeDtypeStruct((GROWS_H, CH), _f32),
        scratch_types=[
            pltpu.VMEM((4, _CHUNK), jnp.int32),
            pltpu.VMEM((4, _CHUNK, CH), _f32),
            pltpu.SemaphoreType.DMA,
        ],
    )
    def k(so_hbm, idx_hbm, out_hbm, idx_v, rows_v, sem):
        wid = lax.axis_index("s") * _NC + lax.axis_index("c")

        def step(p, carry):
            cps = []
            for u in range(4):
                g0 = wid * _RPW + (4 * p + u) * _CHUNK
                pltpu.sync_copy(idx_hbm.at[pl.ds(g0, _CHUNK)], idx_v.at[u])
                cps.append(pltpu.async_copy(so_hbm.at[idx_v.at[u]],
                                            rows_v.at[u], sem))
            for u in range(4):
                g0 = wid * _RPW + (4 * p + u) * _CHUNK
                cps[u].wait()
                pltpu.sync_copy(rows_v.at[u], out_hbm.at[pl.ds(g0, _CHUNK)])
            return carry

        lax.fori_loop(0, _NSTEP // 4, step, 0)

    return k(so_flat, gidx)


def _attn2_body(svin_ref, oe_ref, so_ref):
    sqk = svin_ref[0, :, 0:DH]
    sv = svin_ref[0, :, DH:2 * DH]
    offsets = oe_ref[0, :, 0:NB]
    ends = oe_ref[0, :, NB:2 * NB]

    riota = _fiota((S, 1), 0)
    son = jnp.where((riota >= offsets) & (riota < ends), 1.0, 0.0)

    kn = sqk * (lax.reciprocal(
        jnp.sqrt(jnp.sum(sqk * sqk, axis=1, keepdims=True)) + 1e-6)
        * (1.0 / np.sqrt(DH)))

    si = lax.broadcasted_iota(jnp.int32, (QR, KR), 0)
    li = lax.broadcasted_iota(jnp.int32, (QR, KR), 1)
    cbase = (si // BUCKET) * BUCKET
    band = (li >= cbase) & (li < cbase + 2 * BUCKET)
    selfm = li == si + BUCKET
    stat = jnp.where(selfm, -1e5, jnp.where(band, 0.0, -1e9))
    pad = jnp.zeros((QR, CH - DH - 1), _f32)

    def kseg(arr, g):
        if g == 0:
            return jnp.concatenate([arr[S - BUCKET:S], arr[0:QR]], axis=0)
        return arr[g * QR - BUCKET:g * QR + QR]

    outs = []
    for g in range(NG):
        q = sqk[g * QR:(g + 1) * QR]
        dots = _dot(q, kseg(kn, g), dn=((1,), (1,)))
        eq = _dot(son[g * QR:(g + 1) * QR], kseg(son, g),
                  dn=((1,), (1,)))
        dots = jnp.where(eq > 0.5, dots, -1e9) + stat
        m = jnp.max(dots, axis=1, keepdims=True)
        e = jnp.exp(dots - m)
        se = jnp.sum(e, axis=1, keepdims=True)
        lse = m + jnp.log(se)
        probs = e * lax.reciprocal(se)
        co = _dot(probs, kseg(sv, g))
        outs.append(jnp.concatenate([co, lse, pad], axis=1))
    so_ref[0] = jnp.concatenate(outs, axis=0)


def _attn2(svin, oe2):
    return pl.pallas_call(
        _attn2_body,
        grid=(NTASK_H,),
        in_specs=[
            pl.BlockSpec((1, S, 2 * DH), lambda i: (i, 0, 0)),
            pl.BlockSpec((1, 1, 2 * NB), lambda i: (i, 0, 0)),
        ],
        out_specs=pl.BlockSpec((1, S, CH), lambda i: (i, 0, 0)),
        out_shape=jax.ShapeDtypeStruct((NTASK_H, S, CH), _f32),
        compiler_params=pltpu.CompilerParams(
            dimension_semantics=("arbitrary",)),
    )(svin, oe2)


def _comb_body(ouns_ref, ah_ref):
    o0 = ouns_ref[0, 0:S, 0:DH]
    l0 = ouns_ref[0, 0:S, DH:DH + 1]
    o1 = ouns_ref[0, S:2 * S, 0:DH]
    l1 = ouns_ref[0, S:2 * S, DH:DH + 1]
    mm = jnp.maximum(l0, l1)
    e0 = jnp.exp(l0 - mm)
    e1 = jnp.exp(l1 - mm)
    ah_ref[0, 0] = (o0 * e0 + o1 * e1) * lax.reciprocal(e0 + e1)


def _comb(ouns2):
    return pl.pallas_call(
        _comb_body,
        grid=(H,),
        in_specs=[pl.BlockSpec((1, NHASH * S, CH), lambda i: (i, 0, 0))],
        out_specs=pl.BlockSpec((1, 1, S, DH), lambda i: (0, i, 0, 0)),
        out_shape=jax.ShapeDtypeStruct((1, H, S, DH), _f32),
        compiler_params=pltpu.CompilerParams(
            dimension_semantics=("arbitrary",)),
    )(ouns2)


def kernel(X, missing_mask, W_emb, b_emb, Wqk, Wv, Wo, ln1_g, ln1_b,
           W1, b1, W2, b2, ln2_g, ln2_b, W_out, b_out):
    X2 = X.reshape(RT, F)
    M2 = missing_mask.reshape(RT, F)

    emb = _row_tiled(_emb_body,
                     (('t', F), ('t', F), (F, D), (F, D), (1, D)), (D,))
    x = emb(X2, M2, W_emb[:F], W_emb[F:], b_emb.reshape(1, D))

    rot = jax.random.normal(jax.random.key(42), (L, DH, NHASH, NB // 2), _f32)

    hspec = pl.BlockSpec((1, H, ROWT, DH), lambda i: (i // NST, 0, i % NST, 0))
    qkv = pl.pallas_call(
        _qkv_body,
        grid=(NRT,),
        in_specs=[
            pl.BlockSpec((ROWT, D), lambda i: (i, 0)),
            pl.BlockSpec((D, D), lambda i: (0, 0)),
            pl.BlockSpec((D, D), lambda i: (0, 0)),
        ],
        out_specs=[hspec, hspec],
        out_shape=[jax.ShapeDtypeStruct((B, H, S, DH), _f32)] * 2,
        compiler_params=pltpu.CompilerParams(
            dimension_semantics=("arbitrary",)),
    )
    blk = pl.pallas_call(
        _block_body,
        grid=(NRT,),
        in_specs=[hspec,
                  pl.BlockSpec((ROWT, D), lambda i: (i, 0)),
                  pl.BlockSpec((D, D), lambda i: (0, 0)),
                  pl.BlockSpec((1, D), lambda i: (0, 0)),
                  pl.BlockSpec((1, D), lambda i: (0, 0)),
                  pl.BlockSpec((D, DFF), lambda i: (0, 0)),
                  pl.BlockSpec((1, DFF), lambda i: (0, 0)),
                  pl.BlockSpec((DFF, D), lambda i: (0, 0)),
                  pl.BlockSpec((1, D), lambda i: (0, 0)),
                  pl.BlockSpec((1, D), lambda i: (0, 0)),
                  pl.BlockSpec((1, D), lambda i: (0, 0))],
        out_specs=pl.BlockSpec((ROWT, D), lambda i: (i, 0)),
        out_shape=jax.ShapeDtypeStruct((RT, D), _f32),
        compiler_params=pltpu.CompilerParams(
            dimension_semantics=("arbitrary",)),
    )

    for i in range(L):
        qkh4, vh4 = qkv(x, Wqk[i], Wv[i])
        rot_i = rot[i].transpose(1, 0, 2)

        ah_halves = []
        for b in range(B):
            qkh_b = lax.slice_in_dim(qkh4, b, b + 1, axis=0)
            vh_b = lax.slice_in_dim(vh4, b, b + 1, axis=0)
            vin, guns4, oe4 = _prep(qkh_b, vh_b, rot_i)
            guns = guns4.reshape(GROWS_H)
            svin = _sc_sort(vin.reshape(VROWS_H, 2 * DH), guns)
            so = _attn2(svin.reshape(NTASK_H, S, 2 * DH),
                        oe4.reshape(NTASK_H, 1, 2 * NB))
            ouns = _sc_unsort(so.reshape(GROWS_H, CH), guns)
            ah_halves.append(_comb(ouns.reshape(H, NHASH * S, CH)))
        ah4 = jnp.concatenate(ah_halves, axis=0)

        x = blk(ah4, x, Wo[i], ln1_g[i].reshape(1, D), ln1_b[i].reshape(1, D),
                W1[i], b1[i].reshape(1, DFF), W2[i], b2[i].reshape(1, D),
                ln2_g[i].reshape(1, D), ln2_b[i].reshape(1, D))

    fin = pl.pallas_call(
        _final_body,
        grid=(NRT,),
        in_specs=[
            pl.BlockSpec((ROWT, D), lambda i: (i, 0)),
            pl.BlockSpec((D, F), lambda i: (0, 0)),
            pl.BlockSpec((1, F), lambda i: (0, 0)),
            pl.BlockSpec((ROWT, F), lambda i: (i, 0)),
            pl.BlockSpec((ROWT, F), lambda i: (i, 0)),
        ],
        out_specs=pl.BlockSpec((ROWT, F), lambda i: (i, 0)),
        out_shape=jax.ShapeDtypeStruct((RT, F), _f32),
        compiler_params=pltpu.CompilerParams(
            dimension_semantics=("arbitrary",)),
    )
    out = fin(x, W_out, b_out.reshape(1, F), X2, M2)
    return out.reshape(B, S, F)


# speedup vs baseline: 5.3837x; 1.0439x over previous
import functools

import jax
import jax.numpy as jnp
import numpy as np
from jax import lax
from jax.experimental import pallas as pl
from jax.experimental.pallas import tpu as pltpu
from jax.experimental.pallas import tpu_sc as plsc

B, S, F, D, H, L = 2, 2048, 128, 1024, 16, 2
DH = D // H
DFF = 2048
BUCKET = 64
NHASH = 2
NB = S // BUCKET
RT = B * S
ROWT = 512
NRT = RT // ROWT
NST = S // ROWT
CH = 128
NCB = S // CH
GC = 4
QR = GC * BUCKET
KR = QR + BUCKET
NG = S // QR
NTASK = B * H * NHASH
VROWS = B * H * S
GROWS = NTASK * S
NTASK_H = H * NHASH
VROWS_H = H * S
GROWS_H = NTASK_H * S

_NC, _NS = 2, 16
_NW = _NC * _NS
_CHUNK = 128
_RPW = GROWS_H // _NW
_NSTEP = _RPW // _CHUNK

_f32 = jnp.float32


def _fiota(shape, dim):
    return lax.broadcasted_iota(jnp.int32, shape, dim).astype(_f32)


def _ln(x, g, b):
    m = jnp.mean(x, axis=1, keepdims=True)
    v = jnp.mean((x - m) ** 2, axis=1, keepdims=True)
    return (x - m) * lax.rsqrt(v + 1e-5) * g + b


def _dot(a, b, dn=None):
    if dn is None:
        return lax.dot(a, b, preferred_element_type=_f32)
    return lax.dot_general(a, b, dimension_numbers=(dn, ((), ())),
                           preferred_element_type=_f32)


def _emb_body(x_ref, m_ref, wx_ref, wm_ref, b_ref, o_ref):
    o_ref[...] = (_dot(x_ref[...], wx_ref[...]) + _dot(m_ref[...], wm_ref[...])
                  + b_ref[...])


def _qkv_body(x_ref, wqk_ref, wv_ref, qk_ref, v_ref):
    x = x_ref[...]
    qk = _dot(x, wqk_ref[...])
    v = _dot(x, wv_ref[...])
    for h in range(H):
        qk_ref[0, h] = qk[:, h * DH:(h + 1) * DH]
        v_ref[0, h] = v[:, h * DH:(h + 1) * DH]


def _block_body(a_ref, x_ref, wo_ref, g1_ref, b1g_ref, w1_ref, b1_ref,
                w2_ref, b2_ref, g2_ref, b2g_ref, o_ref):
    a = jnp.concatenate([a_ref[0, h] for h in range(H)], axis=1)
    t = x_ref[...] + _dot(a, wo_ref[...])
    t = _ln(t, g1_ref[...], b1g_ref[...])
    ff = jnp.maximum(_dot(t, w1_ref[...]) + b1_ref[...], 0.0)
    ff = _dot(ff, w2_ref[...]) + b2_ref[...]
    o_ref[...] = _ln(t + ff, g2_ref[...], b2g_ref[...])


def _final_body(x_ref, w_ref, b_ref, X_ref, m_ref, o_ref):
    rec = _dot(x_ref[...], w_ref[...]) + b_ref[...]
    m = m_ref[...]
    o_ref[...] = m * X_ref[...] + (1.0 - m) * rec


def _row_tiled(body, in_widths, out_widths, rows=S):
    in_specs = []
    for w in in_widths:
        if isinstance(w, tuple) and w[0] == 't':
            in_specs.append(pl.BlockSpec((ROWT, w[1]), lambda i: (i, 0)))
        else:
            in_specs.append(pl.BlockSpec(w, lambda i: (0, 0)))
    out_specs = [pl.BlockSpec((ROWT, w), lambda i: (i, 0)) for w in out_widths]
    out_shape = [jax.ShapeDtypeStruct((rows, w), _f32) for w in out_widths]
    return pl.pallas_call(
        body,
        grid=(rows // ROWT,),
        in_specs=in_specs,
        out_specs=out_specs if len(out_widths) > 1 else out_specs[0],
        out_shape=out_shape if len(out_widths) > 1 else out_shape[0],
        compiler_params=pltpu.CompilerParams(
            dimension_semantics=("arbitrary",)),
    )


def _prep_body(qk_ref, v_ref, rot_ref, vin_ref, guns_ref, oe_ref):
    pid = pl.program_id(0)
    qk = qk_ref[0, 0]
    v = v_ref[0, 0]
    vin_ref[0] = jnp.concatenate([qk, v], axis=1)

    riota = _fiota((S, 1), 0)
    r128 = _fiota((CH, CH), 0)
    c128 = _fiota((CH, CH), 1)
    Ls = jnp.where(c128 < r128, 1.0, 0.0)
    r32 = _fiota((NB, NB), 0)
    c32 = _fiota((NB, NB), 1)
    Ts32 = jnp.where(r32 < c32, 1.0, 0.0)

    for r in range(NHASH):
        rotated = _dot(qk, rot_ref[r])
        full = jnp.concatenate([rotated, -rotated], axis=1)
        mx = jnp.max(full, axis=1, keepdims=True)
        iota_b = _fiota((S, NB), 1)
        bucket = jnp.min(jnp.where(full >= mx, iota_b, 1e9), axis=1,
                         keepdims=True)
        onehot = jnp.where(iota_b == bucket, 1.0, 0.0)
        blkall = jnp.concatenate(
            [onehot[c * CH:(c + 1) * CH] for c in range(NCB)], axis=1)
        exall = _dot(Ls, blkall)
        lastrow = exall[CH - 1:CH, :] + blkall[CH - 1:CH, :]
        wbps = []
        run = jnp.zeros((1, NB), _f32)
        for c in range(NCB):
            sl = slice(c * NB, (c + 1) * NB)
            wbps.append(exall[:, sl] + run)
            run = run + lastrow[:, sl]
        wbp = jnp.concatenate(wbps, axis=0)
        totals = run
        offsets = _dot(totals, Ts32)
        rank = jnp.sum(onehot * (wbp + offsets), axis=1, keepdims=True)

        base = (pid * NHASH + r) * S
        guns_ref[0, r] = (rank + base.astype(_f32)).astype(jnp.int32)
        oe_ref[0, r] = jnp.concatenate([offsets, offsets + totals], axis=1)


def _prep(qkh, vh, rot_i):
    return pl.pallas_call(
        _prep_body,
        grid=(H,),
        in_specs=[
            pl.BlockSpec((1, 1, S, DH), lambda i: (0, i, 0, 0)),
            pl.BlockSpec((1, 1, S, DH), lambda i: (0, i, 0, 0)),
            pl.BlockSpec((NHASH, DH, NB // 2), lambda i: (0, 0, 0)),
        ],
        out_specs=[
            pl.BlockSpec((1, S, 2 * DH), lambda i: (i, 0, 0)),
            pl.BlockSpec((1, NHASH, S, 1), lambda i: (i, 0, 0, 0)),
            pl.BlockSpec((1, NHASH, 1, 2 * NB), lambda i: (i, 0, 0, 0)),
        ],
        out_shape=[
            jax.ShapeDtypeStruct((H, S, 2 * DH), _f32),
            jax.ShapeDtypeStruct((H, NHASH, S, 1), jnp.int32),
            jax.ShapeDtypeStruct((H, NHASH, 1, 2 * NB), _f32),
        ],
        compiler_params=pltpu.CompilerParams(
            dimension_semantics=("arbitrary",)),
    )(qkh, vh, rot_i)


def _sc_sort(vin_flat, gidx):
    @functools.partial(
        pl.kernel,
        mesh=plsc.VectorSubcoreMesh(core_axis_name="c", subcore_axis_name="s"),
        out_type=jax.ShapeDtypeStruct((GROWS_H, 2 * DH), _f32),
        scratch_types=[
            pltpu.VMEM((4, _CHUNK), jnp.int32),
            pltpu.VMEM((4, _CHUNK, 2 * DH), _f32),
            pltpu.SemaphoreType.DMA,
        ],
    )
    def k(vin_hbm, idx_hbm, out_hbm, idx_v, rows_v, sem):
        wid = lax.axis_index("s") * _NC + lax.axis_index("c")

        def step(p, carry):
            cps = []
            for u in range(4):
                g0 = wid * _RPW + (4 * p + u) * _CHUNK
                t = g0 // S
                src0 = (t // NHASH) * S + (g0 - t * S)
                pltpu.sync_copy(vin_hbm.at[pl.ds(src0, _CHUNK)], rows_v.at[u])
                pltpu.sync_copy(idx_hbm.at[pl.ds(g0, _CHUNK)], idx_v.at[u])
                cps.append(pltpu.async_copy(rows_v.at[u],
                                            out_hbm.at[idx_v.at[u]], sem))
            for cp in cps:
                cp.wait()
            return carry

        lax.fori_loop(0, _NSTEP // 4, step, 0)

    return k(vin_flat, gidx)


def _sc_unsort(so_flat, gidx):
    @functools.partial(
        pl.kernel,
        mesh=plsc.VectorSubcoreMesh(core_axis_name="c", subcore_axis_name="s"),
        out_type=jax.ShapeDtypeStruct((GROWS_H, CH), _f32),
        scratch_types=[
            pltpu.VMEM((4, _CHUNK), jnp.int32),
            pltpu.VMEM((4, _CHUNK, CH), _f32),
            pltpu.SemaphoreType.DMA,
        ],
    )
    def k(so_hbm, idx_hbm, out_hbm, idx_v, rows_v, sem):
        wid = lax.axis_index("s") * _NC + lax.axis_index("c")

        def step(p, carry):
            cps = []
            for u in range(4):
                g0 = wid * _RPW + (4 * p + u) * _CHUNK
                pltpu.sync_copy(idx_hbm.at[pl.ds(g0, _CHUNK)], idx_v.at[u])
                cps.append(pltpu.async_copy(so_hbm.at[idx_v.at[u]],
                                            rows_v.at[u], sem))
            for u in range(4):
                g0 = wid * _RPW + (4 * p + u) * _CHUNK
                cps[u].wait()
                pltpu.sync_copy(rows_v.at[u], out_hbm.at[pl.ds(g0, _CHUNK)])
            return carry

        lax.fori_loop(0, _NSTEP // 4, step, 0)

    return k(so_flat, gidx)


def _attn2_body(svin_ref, oe_ref, so_ref):
    sqk = svin_ref[0, :, 0:DH]
    sv = svin_ref[0, :, DH:2 * DH]
    offsets = oe_ref[0, :, 0:NB]
    ends = oe_ref[0, :, NB:2 * NB]

    riota = _fiota((S, 1), 0)
    son = jnp.where((riota >= offsets) & (riota < ends), 1.0, 0.0)

    kn = sqk * (lax.reciprocal(
        jnp.sqrt(jnp.sum(sqk * sqk, axis=1, keepdims=True)) + 1e-6)
        * (1.0 / np.sqrt(DH)))

    si = lax.broadcasted_iota(jnp.int32, (QR, KR), 0)
    li = lax.broadcasted_iota(jnp.int32, (QR, KR), 1)
    cbase = (si // BUCKET) * BUCKET
    band = (li >= cbase) & (li < cbase + 2 * BUCKET)
    selfm = li == si + BUCKET
    stat = jnp.where(selfm, -1e5, jnp.where(band, 0.0, -1e9))
    pad = jnp.zeros((QR, CH - DH - 1), _f32)

    def kseg(arr, g):
        if g == 0:
            return jnp.concatenate([arr[S - BUCKET:S], arr[0:QR]], axis=0)
        return arr[g * QR - BUCKET:g * QR + QR]

    outs = []
    for g in range(NG):
        q = sqk[g * QR:(g + 1) * QR]
        dots = _dot(q, kseg(kn, g), dn=((1,), (1,)))
        eq = _dot(son[g * QR:(g + 1) * QR], kseg(son, g),
                  dn=((1,), (1,)))
        dots = jnp.where(eq > 0.5, dots, -1e9) + stat
        m = jnp.max(dots, axis=1, keepdims=True)
        e = jnp.exp(dots - m)
        se = jnp.sum(e, axis=1, keepdims=True)
        lse = m + jnp.log(se)
        probs = e * lax.reciprocal(se)
        co = _dot(probs, kseg(sv, g))
        outs.append(jnp.concatenate([co, lse, pad], axis=1))
    so_ref[0] = jnp.concatenate(outs, axis=0)


def _attn2(svin, oe2):
    return pl.pallas_call(
        _attn2_body,
        grid=(NTASK_H,),
        in_specs=[
            pl.BlockSpec((1, S, 2 * DH), lambda i: (i, 0, 0)),
            pl.BlockSpec((1, 1, 2 * NB), lambda i: (i, 0, 0)),
        ],
        out_specs=pl.BlockSpec((1, S, CH), lambda i: (i, 0, 0)),
        out_shape=jax.ShapeDtypeStruct((NTASK_H, S, CH), _f32),
        compiler_params=pltpu.CompilerParams(
            dimension_semantics=("arbitrary",)),
    )(svin, oe2)


def _comb_body(ouns_ref, ah_ref):
    o0 = ouns_ref[0, 0:S, 0:DH]
    l0 = ouns_ref[0, 0:S, DH:DH + 1]
    o1 = ouns_ref[0, S:2 * S, 0:DH]
    l1 = ouns_ref[0, S:2 * S, DH:DH + 1]
    mm = jnp.maximum(l0, l1)
    e0 = jnp.exp(l0 - mm)
    e1 = jnp.exp(l1 - mm)
    ah_ref[0, 0] = (o0 * e0 + o1 * e1) * lax.reciprocal(e0 + e1)


def _comb(ouns2):
    return pl.pallas_call(
        _comb_body,
        grid=(H,),
        in_specs=[pl.BlockSpec((1, NHASH * S, CH), lambda i: (i, 0, 0))],
        out_specs=pl.BlockSpec((1, 1, S, DH), lambda i: (0, i, 0, 0)),
        out_shape=jax.ShapeDtypeStruct((1, H, S, DH), _f32),
        compiler_params=pltpu.CompilerParams(
            dimension_semantics=("arbitrary",)),
    )(ouns2)


def kernel(X, missing_mask, W_emb, b_emb, Wqk, Wv, Wo, ln1_g, ln1_b,
           W1, b1, W2, b2, ln2_g, ln2_b, W_out, b_out):
    rot = jax.random.normal(jax.random.key(42), (L, DH, NHASH, NB // 2), _f32)

    emb = _row_tiled(_emb_body,
                     (('t', F), ('t', F), (F, D), (F, D), (1, D)), (D,))
    hspec = pl.BlockSpec((1, H, ROWT, DH), lambda i: (0, 0, i, 0))
    qkv = pl.pallas_call(
        _qkv_body,
        grid=(NST,),
        in_specs=[
            pl.BlockSpec((ROWT, D), lambda i: (i, 0)),
            pl.BlockSpec((D, D), lambda i: (0, 0)),
            pl.BlockSpec((D, D), lambda i: (0, 0)),
        ],
        out_specs=[hspec, hspec],
        out_shape=[jax.ShapeDtypeStruct((1, H, S, DH), _f32)] * 2,
        compiler_params=pltpu.CompilerParams(
            dimension_semantics=("arbitrary",)),
    )
    blk = pl.pallas_call(
        _block_body,
        grid=(NST,),
        in_specs=[hspec,
                  pl.BlockSpec((ROWT, D), lambda i: (i, 0)),
                  pl.BlockSpec((D, D), lambda i: (0, 0)),
                  pl.BlockSpec((1, D), lambda i: (0, 0)),
                  pl.BlockSpec((1, D), lambda i: (0, 0)),
                  pl.BlockSpec((D, DFF), lambda i: (0, 0)),
                  pl.BlockSpec((1, DFF), lambda i: (0, 0)),
                  pl.BlockSpec((DFF, D), lambda i: (0, 0)),
                  pl.BlockSpec((1, D), lambda i: (0, 0)),
                  pl.BlockSpec((1, D), lambda i: (0, 0)),
                  pl.BlockSpec((1, D), lambda i: (0, 0))],
        out_specs=pl.BlockSpec((ROWT, D), lambda i: (i, 0)),
        out_shape=jax.ShapeDtypeStruct((S, D), _f32),
        compiler_params=pltpu.CompilerParams(
            dimension_semantics=("arbitrary",)),
    )
    fin = pl.pallas_call(
        _final_body,
        grid=(NST,),
        in_specs=[
            pl.BlockSpec((ROWT, D), lambda i: (i, 0)),
            pl.BlockSpec((D, F), lambda i: (0, 0)),
            pl.BlockSpec((1, F), lambda i: (0, 0)),
            pl.BlockSpec((ROWT, F), lambda i: (i, 0)),
            pl.BlockSpec((ROWT, F), lambda i: (i, 0)),
        ],
        out_specs=pl.BlockSpec((ROWT, F), lambda i: (i, 0)),
        out_shape=jax.ShapeDtypeStruct((S, F), _f32),
        compiler_params=pltpu.CompilerParams(
            dimension_semantics=("arbitrary",)),
    )

    xs = []
    for b in range(B):
        xs.append(emb(X[b], missing_mask[b], W_emb[:F], W_emb[F:],
                      b_emb.reshape(1, D)))

    for i in range(L):
        rot_i = rot[i].transpose(1, 0, 2)
        for b in range(B):
            qkh4, vh4 = qkv(xs[b], Wqk[i], Wv[i])
            vin, guns4, oe4 = _prep(qkh4, vh4, rot_i)
            guns = guns4.reshape(GROWS_H)
            svin = _sc_sort(vin.reshape(VROWS_H, 2 * DH), guns)
            so = _attn2(svin.reshape(NTASK_H, S, 2 * DH),
                        oe4.reshape(NTASK_H, 1, 2 * NB))
            ouns = _sc_unsort(so.reshape(GROWS_H, CH), guns)
            ah4 = _comb(ouns.reshape(H, NHASH * S, CH))
            xs[b] = blk(ah4, xs[b], Wo[i], ln1_g[i].reshape(1, D),
                        ln1_b[i].reshape(1, D), W1[i], b1[i].reshape(1, DFF),
                        W2[i], b2[i].reshape(1, D), ln2_g[i].reshape(1, D),
                        ln2_b[i].reshape(1, D))

    outs = [fin(xs[b], W_out, b_out.reshape(1, F), X[b], missing_mask[b])
            for b in range(B)]
    return jnp.stack(outs, axis=0)
